# baseline (speedup 1.0000x reference)
import functools

import jax
import jax.numpy as jnp
from jax.experimental import pallas as pl
from jax.experimental.pallas import tpu as pltpu

F32 = jnp.float32


def _mm_body(x_ref, y_ref, o_ref, *, lowp):
    @pl.when(pl.program_id(2) == 0)
    def _init():
        o_ref[...] = jnp.zeros_like(o_ref)

    x, y = x_ref[...], y_ref[...]
    if lowp:
        x, y = x.astype(jnp.bfloat16), y.astype(jnp.bfloat16)
    o_ref[...] += jnp.dot(x, y, preferred_element_type=F32)


def _mm(a, b, bm=None, bk=None, bn=None, n_out=None, lowp=True):
    m, k = a.shape
    k2, n = b.shape
    assert k == k2, (a.shape, b.shape)
    bm = min(512, m) if bm is None else bm
    bk = min(1024, k) if bk is None else bk
    bn = min(512, n) if bn is None else bn
    assert m % bm == 0 and k % bk == 0 and n % bn == 0, (a.shape, b.shape, bm, bk, bn)
    n_out = n if n_out is None else n_out
    return pl.pallas_call(
        functools.partial(_mm_body, lowp=lowp),
        grid=(m // bm, n // bn, k // bk),
        in_specs=[
            pl.BlockSpec((bm, bk), lambda i, j, kk: (i, kk)),
            pl.BlockSpec((bk, bn), lambda i, j, kk: (kk, j)),
        ],
        out_specs=pl.BlockSpec((bm, bn), lambda i, j, kk: (i, j)),
        out_shape=jax.ShapeDtypeStruct((m, n_out), F32),
        compiler_params=pltpu.CompilerParams(
            dimension_semantics=("parallel", "parallel", "arbitrary")),
    )(a, b)


def _att_body(e1s_ref, e1f_ref, e1a_ref, e2s_ref, e2f_ref, e2a_ref,
              w1_ref, u1_ref, w2_ref, u2_ref, wc_ref, uc_ref,
              l1_ref, l2_ref, co_ref, a1_ref, a2_ref, ac_ref):
    def score(e, w, u_t):
        v = jnp.tanh(jnp.dot(e, w, preferred_element_type=F32))
        return jnp.sum(v * u_t, axis=1, keepdims=True)

    def att3(es, ef, ea, w, u_t):
        ss, sf, sa = score(es, w, u_t), score(ef, w, u_t), score(ea, w, u_t)
        mx = jnp.maximum(jnp.maximum(ss, sf), sa)
        xs, xf, xa = jnp.exp(ss - mx), jnp.exp(sf - mx), jnp.exp(sa - mx)
        den = xs + xf + xa
        als, alf, ala = xs / den, xf / den, xa / den
        l = als * es + alf * ef + ala * ea
        return l, jnp.concatenate([als, alf, ala], axis=1)

    l1, a1 = att3(e1s_ref[...], e1f_ref[...], e1a_ref[...], w1_ref[...], u1_ref[...])
    l2, a2 = att3(e2s_ref[...], e2f_ref[...], e2a_ref[...], w2_ref[...], u2_ref[...])
    s1 = score(l1, wc_ref[...], uc_ref[...])
    s2 = score(l2, wc_ref[...], uc_ref[...])
    mx = jnp.maximum(s1, s2)
    x1, x2 = jnp.exp(s1 - mx), jnp.exp(s2 - mx)
    den = x1 + x2
    b1, b2 = x1 / den, x2 / den
    l1_ref[...] = l1
    l2_ref[...] = l2
    co_ref[...] = b1 * l1 + b2 * l2
    a1_ref[...] = a1
    a2_ref[...] = a2
    ac_ref[...] = jnp.concatenate([b1, b2], axis=1)


def _attention(e1s, e1f, e1a, e2s, e2f, e2a, w1, u1, w2, u2, wc, uc):
    n, h = e1s.shape
    bm = min(512, n)
    row = lambda i: (i, 0)
    fixed = lambda i: (0, 0)
    eb = pl.BlockSpec((bm, h), row)
    wb = pl.BlockSpec((h, h), fixed)
    ub = pl.BlockSpec((1, h), fixed)
    return pl.pallas_call(
        _att_body,
        grid=(n // bm,),
        in_specs=[eb, eb, eb, eb, eb, eb, wb, ub, wb, ub, wb, ub],
        out_specs=[
            pl.BlockSpec((bm, h), row),
            pl.BlockSpec((bm, h), row),
            pl.BlockSpec((bm, h), row),
            pl.BlockSpec((bm, 3), row),
            pl.BlockSpec((bm, 3), row),
            pl.BlockSpec((bm, 2), row),
        ],
        out_shape=[
            jax.ShapeDtypeStruct((n, h), F32),
            jax.ShapeDtypeStruct((n, h), F32),
            jax.ShapeDtypeStruct((n, h), F32),
            jax.ShapeDtypeStruct((n, 3), F32),
            jax.ShapeDtypeStruct((n, 3), F32),
            jax.ShapeDtypeStruct((n, 2), F32),
        ],
        compiler_params=pltpu.CompilerParams(
            dimension_semantics=("parallel",)),
    )(e1s, e1f, e1a, e2s, e2f, e2a, w1, u1.T, w2, u2.T, wc, uc.T)


def _pad_cols(x, mult=128):
    d = x.shape[1]
    p = (-d) % mult
    return jnp.pad(x, ((0, 0), (0, p))) if p else x


def _pad_rows(x, mult=128):
    d = x.shape[0]
    p = (-d) % mult
    return jnp.pad(x, ((0, p), (0, 0))) if p else x


def kernel(features_omics1, features_omics2, adj_spatial_omics1,
           adj_feature_omics1, adj_augmented_omics1, adj_spatial_omics2,
           adj_feature_omics2, adj_augmented_omics2, W_enc1_sp, W_enc1_ft,
           W_enc1_aug, W_enc2_sp, W_enc2_ft, W_enc2_aug, W_dec1, W_dec2,
           att1_w, att1_u, att2_w, att2_u, attc_w, attc_u):
    h = W_enc1_sp.shape[1]
    d1 = W_dec1.shape[1]

    a1s, a1f, a1a = adj_spatial_omics1, adj_feature_omics1, adj_augmented_omics1
    a2s, a2f, a2a = adj_spatial_omics2, adj_feature_omics2, adj_augmented_omics2

    x1p = _pad_cols(features_omics1)
    w1c = _pad_rows(jnp.concatenate([W_enc1_sp, W_enc1_ft, W_enc1_aug], axis=1))
    w2c = jnp.concatenate([W_enc2_sp, W_enc2_ft, W_enc2_aug], axis=1)
    y1 = _mm(x1p, w1c)
    y2 = _mm(features_omics2, w2c)

    e1s = _mm(a1s, y1[:, 0 * h:1 * h])
    e1f = _mm(a1f, y1[:, 1 * h:2 * h])
    e1a = _mm(a1a, y1[:, 2 * h:3 * h])
    e2s = _mm(a2s, y2[:, 0 * h:1 * h])
    e2f = _mm(a2f, y2[:, 1 * h:2 * h])
    e2a = _mm(a2a, y2[:, 2 * h:3 * h])

    l1, l2, comb, al1, al2, alc = _attention(
        e1s, e1f, e1a, e2s, e2f, e2a,
        att1_w, att1_u, att2_w, att2_u, attc_w, attc_u)

    wd1p = _pad_cols(W_dec1)
    md1 = _mm(wd1p, _pad_rows(W_enc1_sp), bm=h, lowp=False)
    md2 = _mm(W_dec2, W_enc2_sp, bm=h, lowp=False)

    u1 = _mm(a1s, jnp.concatenate([comb, l2], axis=1))
    u2 = _mm(a2s, jnp.concatenate([comb, l1], axis=1))

    rec1 = _mm(u1[:, :h], wd1p, n_out=d1)
    rec2 = _mm(u2[:, :h], W_dec2)
    x2r = _mm(a1s, _mm(u1[:, h:], md1, lowp=False))
    x1r = _mm(a2s, _mm(u2[:, h:], md2, lowp=False))

    return (l1, l2, comb, rec1, rec2, x1r, x2r, al1, al2, alc,
            e1s, e1f, e1a, e2s, e2f, e2a)


# speedup vs baseline: 1.5016x; 1.5016x over previous
import functools

import jax
import jax.numpy as jnp
from jax.experimental import pallas as pl
from jax.experimental.pallas import tpu as pltpu

F32 = jnp.float32
BF16 = jnp.bfloat16


def _dot(x, y):
    return jnp.dot(x.astype(BF16), y.astype(BF16), preferred_element_type=F32)


def _proj_body(x_ref, wc_ref, wd_ref, wsp_ref, y_ref, md_ref):
    @pl.when(pl.program_id(0) == 0)
    def _():
        md_ref[...] = jnp.dot(wd_ref[...], wsp_ref[...],
                              preferred_element_type=F32)

    y_ref[...] = _dot(x_ref[...], wc_ref[...])


def _proj(x, wc, wd, wsp, bm=512):
    n, d = x.shape
    kn = wc.shape[1]
    h = wd.shape[0]
    return pl.pallas_call(
        _proj_body,
        grid=(n // bm,),
        in_specs=[
            pl.BlockSpec((bm, d), lambda i: (i, 0)),
            pl.BlockSpec((d, kn), lambda i: (0, 0)),
            pl.BlockSpec((h, d), lambda i: (0, 0)),
            pl.BlockSpec((d, h), lambda i: (0, 0)),
        ],
        out_specs=[
            pl.BlockSpec((bm, kn), lambda i: (i, 0)),
            pl.BlockSpec((h, h), lambda i: (0, 0)),
        ],
        out_shape=[
            jax.ShapeDtypeStruct((n, kn), F32),
            jax.ShapeDtypeStruct((h, h), F32),
        ],
        compiler_params=pltpu.CompilerParams(
            dimension_semantics=("arbitrary",)),
    )(x, wc, wd, wsp)


def _agg6_body(*refs):
    a_refs, y_refs, o_refs = refs[0:6], refs[6:12], refs[12:18]

    @pl.when(pl.program_id(1) == 0)
    def _():
        for o in o_refs:
            o[...] = jnp.zeros_like(o)

    for a, y, o in zip(a_refs, y_refs, o_refs):
        o[...] += _dot(a[...], y[...])


def _agg6(adjs, y1, y2, h, bm=512, bk=1024):
    n = adjs[0].shape[0]
    adj_spec = pl.BlockSpec((bm, bk), lambda i, k: (i, k))
    y_specs = [pl.BlockSpec((bk, h), lambda i, k, t=t: (k, t))
               for t in range(3)]
    out_spec = pl.BlockSpec((bm, h), lambda i, k: (i, 0))
    return pl.pallas_call(
        _agg6_body,
        grid=(n // bm, n // bk),
        in_specs=[adj_spec] * 6 + y_specs + y_specs,
        out_specs=[out_spec] * 6,
        out_shape=[jax.ShapeDtypeStruct((n, h), F32)] * 6,
        compiler_params=pltpu.CompilerParams(
            dimension_semantics=("parallel", "arbitrary")),
    )(*adjs, y1, y1, y1, y2, y2, y2)


def _att_body(e1s_ref, e1f_ref, e1a_ref, e2s_ref, e2f_ref, e2a_ref,
              w1_ref, u1_ref, w2_ref, u2_ref, wc_ref, uc_ref,
              l1_ref, l2_ref, co_ref, a1_ref, a2_ref, ac_ref):
    def score(e, w, u_t):
        v = jnp.tanh(jnp.dot(e, w, preferred_element_type=F32))
        return jnp.sum(v * u_t, axis=1, keepdims=True)

    def att3(es, ef, ea, w, u_t):
        ss, sf, sa = score(es, w, u_t), score(ef, w, u_t), score(ea, w, u_t)
        mx = jnp.maximum(jnp.maximum(ss, sf), sa)
        xs, xf, xa = jnp.exp(ss - mx), jnp.exp(sf - mx), jnp.exp(sa - mx)
        den = xs + xf + xa
        als, alf, ala = xs / den, xf / den, xa / den
        l = als * es + alf * ef + ala * ea
        return l, jnp.concatenate([als, alf, ala], axis=1)

    l1, a1 = att3(e1s_ref[...], e1f_ref[...], e1a_ref[...],
                  w1_ref[...], u1_ref[...])
    l2, a2 = att3(e2s_ref[...], e2f_ref[...], e2a_ref[...],
                  w2_ref[...], u2_ref[...])
    s1 = score(l1, wc_ref[...], uc_ref[...])
    s2 = score(l2, wc_ref[...], uc_ref[...])
    mx = jnp.maximum(s1, s2)
    x1, x2 = jnp.exp(s1 - mx), jnp.exp(s2 - mx)
    den = x1 + x2
    b1, b2 = x1 / den, x2 / den
    l1_ref[...] = l1
    l2_ref[...] = l2
    co_ref[...] = b1 * l1 + b2 * l2
    a1_ref[...] = a1
    a2_ref[...] = a2
    ac_ref[...] = jnp.concatenate([b1, b2], axis=1)


def _attention(e1s, e1f, e1a, e2s, e2f, e2a, w1, u1, w2, u2, wc, uc):
    n, h = e1s.shape
    bm = min(512, n)
    row = lambda i: (i, 0)
    fixed = lambda i: (0, 0)
    eb = pl.BlockSpec((bm, h), row)
    wb = pl.BlockSpec((h, h), fixed)
    ub = pl.BlockSpec((1, h), fixed)
    return pl.pallas_call(
        _att_body,
        grid=(n // bm,),
        in_specs=[eb, eb, eb, eb, eb, eb, wb, ub, wb, ub, wb, ub],
        out_specs=[
            pl.BlockSpec((bm, h), row),
            pl.BlockSpec((bm, h), row),
            pl.BlockSpec((bm, h), row),
            pl.BlockSpec((bm, 3), row),
            pl.BlockSpec((bm, 3), row),
            pl.BlockSpec((bm, 2), row),
        ],
        out_shape=[
            jax.ShapeDtypeStruct((n, h), F32),
            jax.ShapeDtypeStruct((n, h), F32),
            jax.ShapeDtypeStruct((n, h), F32),
            jax.ShapeDtypeStruct((n, 3), F32),
            jax.ShapeDtypeStruct((n, 3), F32),
            jax.ShapeDtypeStruct((n, 2), F32),
        ],
        compiler_params=pltpu.CompilerParams(
            dimension_semantics=("parallel",)),
    )(e1s, e1f, e1a, e2s, e2f, e2a, w1, u1.T, w2, u2.T, wc, uc.T)


def _u4_body(a1_ref, a2_ref, cb_ref, l1_ref, l2_ref,
             u1c_ref, u1l_ref, u2c_ref, u2l_ref):
    @pl.when(pl.program_id(1) == 0)
    def _():
        for o in (u1c_ref, u1l_ref, u2c_ref, u2l_ref):
            o[...] = jnp.zeros_like(o)

    a1 = a1_ref[...].astype(BF16)
    a2 = a2_ref[...].astype(BF16)
    cb = cb_ref[...].astype(BF16)
    u1c_ref[...] += jnp.dot(a1, cb, preferred_element_type=F32)
    u1l_ref[...] += jnp.dot(a1, l2_ref[...].astype(BF16),
                            preferred_element_type=F32)
    u2c_ref[...] += jnp.dot(a2, cb, preferred_element_type=F32)
    u2l_ref[...] += jnp.dot(a2, l1_ref[...].astype(BF16),
                            preferred_element_type=F32)


def _u4(a1s, a2s, comb, l1, l2, bm=512, bk=1024):
    n, h = comb.shape
    adj_spec = pl.BlockSpec((bm, bk), lambda i, k: (i, k))
    vec_spec = pl.BlockSpec((bk, h), lambda i, k: (k, 0))
    out_spec = pl.BlockSpec((bm, h), lambda i, k: (i, 0))
    return pl.pallas_call(
        _u4_body,
        grid=(n // bm, n // bk),
        in_specs=[adj_spec, adj_spec, vec_spec, vec_spec, vec_spec],
        out_specs=[out_spec] * 4,
        out_shape=[jax.ShapeDtypeStruct((n, h), F32)] * 4,
        compiler_params=pltpu.CompilerParams(
            dimension_semantics=("parallel", "arbitrary")),
    )(a1s, a2s, comb, l1, l2)


def _rec_body(u1_ref, u2_ref, wd1_ref, wd2_ref, r1_ref, r2_ref):
    r1_ref[...] = _dot(u1_ref[...], wd1_ref[...])
    r2_ref[...] = _dot(u2_ref[...], wd2_ref[...])


def _rec(u1c, u2c, wd1, wd2, bm=512):
    n, h = u1c.shape
    d1 = wd1.shape[1]
    d2 = wd2.shape[1]
    return pl.pallas_call(
        _rec_body,
        grid=(n // bm,),
        in_specs=[
            pl.BlockSpec((bm, h), lambda i: (i, 0)),
            pl.BlockSpec((bm, h), lambda i: (i, 0)),
            pl.BlockSpec((h, d1), lambda i: (0, 0)),
            pl.BlockSpec((h, d2), lambda i: (0, 0)),
        ],
        out_specs=[
            pl.BlockSpec((bm, d1), lambda i: (i, 0)),
            pl.BlockSpec((bm, d2), lambda i: (i, 0)),
        ],
        out_shape=[
            jax.ShapeDtypeStruct((n, d1), F32),
            jax.ShapeDtypeStruct((n, d2), F32),
        ],
        compiler_params=pltpu.CompilerParams(
            dimension_semantics=("arbitrary",)),
    )(u1c, u2c, wd1, wd2)


def _xr_body(a1_ref, a2_ref, u1l_ref, u2l_ref, md1_ref, md2_ref,
             x2_ref, x1_ref):
    @pl.when(pl.program_id(1) == 0)
    def _():
        x2_ref[...] = jnp.zeros_like(x2_ref)
        x1_ref[...] = jnp.zeros_like(x1_ref)

    z2 = _dot(u1l_ref[...], md1_ref[...])
    z1 = _dot(u2l_ref[...], md2_ref[...])
    x2_ref[...] += _dot(a1_ref[...], z2)
    x1_ref[...] += _dot(a2_ref[...], z1)


def _xr(a1s, a2s, u1l, u2l, md1, md2, bm=512, bk=1024):
    n, h = u1l.shape
    adj_spec = pl.BlockSpec((bm, bk), lambda i, k: (i, k))
    vec_spec = pl.BlockSpec((bk, h), lambda i, k: (k, 0))
    md_spec = pl.BlockSpec((h, h), lambda i, k: (0, 0))
    out_spec = pl.BlockSpec((bm, h), lambda i, k: (i, 0))
    return pl.pallas_call(
        _xr_body,
        grid=(n // bm, n // bk),
        in_specs=[adj_spec, adj_spec, vec_spec, vec_spec, md_spec, md_spec],
        out_specs=[out_spec] * 2,
        out_shape=[jax.ShapeDtypeStruct((n, h), F32)] * 2,
        compiler_params=pltpu.CompilerParams(
            dimension_semantics=("parallel", "arbitrary")),
    )(a1s, a2s, u1l, u2l, md1, md2)


def kernel(features_omics1, features_omics2, adj_spatial_omics1,
           adj_feature_omics1, adj_augmented_omics1, adj_spatial_omics2,
           adj_feature_omics2, adj_augmented_omics2, W_enc1_sp, W_enc1_ft,
           W_enc1_aug, W_enc2_sp, W_enc2_ft, W_enc2_aug, W_dec1, W_dec2,
           att1_w, att1_u, att2_w, att2_u, attc_w, attc_u):
    n = features_omics1.shape[0]
    h = W_enc1_sp.shape[1]
    bm = min(512, n)
    bk = min(1024, n)

    a1s = adj_spatial_omics1
    a2s = adj_spatial_omics2

    w1c = jnp.concatenate([W_enc1_sp, W_enc1_ft, W_enc1_aug], axis=1)
    w2c = jnp.concatenate([W_enc2_sp, W_enc2_ft, W_enc2_aug], axis=1)
    y1, md1 = _proj(features_omics1, w1c, W_dec1, W_enc1_sp, bm=bm)
    y2, md2 = _proj(features_omics2, w2c, W_dec2, W_enc2_sp, bm=bm)

    e1s, e1f, e1a, e2s, e2f, e2a = _agg6(
        (a1s, adj_feature_omics1, adj_augmented_omics1,
         a2s, adj_feature_omics2, adj_augmented_omics2),
        y1, y2, h, bm=bm, bk=bk)

    l1, l2, comb, al1, al2, alc = _attention(
        e1s, e1f, e1a, e2s, e2f, e2a,
        att1_w, att1_u, att2_w, att2_u, attc_w, attc_u)

    u1c, u1l, u2c, u2l = _u4(a1s, a2s, comb, l1, l2, bm=bm, bk=bk)
    rec1, rec2 = _rec(u1c, u2c, W_dec1, W_dec2, bm=bm)
    x2r, x1r = _xr(a1s, a2s, u1l, u2l, md1, md2, bm=bm, bk=bk)

    return (l1, l2, comb, rec1, rec2, x1r, x2r, al1, al2, alc,
            e1s, e1f, e1a, e2s, e2f, e2a)


# speedup vs baseline: 1.5463x; 1.0298x over previous
import jax
import jax.numpy as jnp
from jax.experimental import pallas as pl
from jax.experimental.pallas import tpu as pltpu

F32 = jnp.float32
BF16 = jnp.bfloat16


def _dot(x, y):
    return jnp.dot(x.astype(BF16), y.astype(BF16), preferred_element_type=F32)


def _proj_body(x1_ref, x2_ref, w1c_ref, w2c_ref, wd1_ref, w1sp_ref,
               wd2_ref, w2sp_ref, y1_ref, y2_ref, md1_ref, md2_ref):
    @pl.when(pl.program_id(0) == 0)
    def _():
        md1_ref[...] = jnp.dot(wd1_ref[...], w1sp_ref[...],
                               preferred_element_type=F32)
        md2_ref[...] = jnp.dot(wd2_ref[...], w2sp_ref[...],
                               preferred_element_type=F32)

    y1_ref[...] = _dot(x1_ref[...], w1c_ref[...])
    y2_ref[...] = _dot(x2_ref[...], w2c_ref[...])


def _proj(x1, x2, w1c, w2c, wd1, w1sp, wd2, w2sp, bm):
    n, d1 = x1.shape
    d2 = x2.shape[1]
    kn = w1c.shape[1]
    h = wd1.shape[0]
    fixed = lambda i: (0, 0)
    return pl.pallas_call(
        _proj_body,
        grid=(n // bm,),
        in_specs=[
            pl.BlockSpec((bm, d1), lambda i: (i, 0)),
            pl.BlockSpec((bm, d2), lambda i: (i, 0)),
            pl.BlockSpec((d1, kn), fixed),
            pl.BlockSpec((d2, kn), fixed),
            pl.BlockSpec((h, d1), fixed),
            pl.BlockSpec((d1, h), fixed),
            pl.BlockSpec((h, d2), fixed),
            pl.BlockSpec((d2, h), fixed),
        ],
        out_specs=[
            pl.BlockSpec((bm, kn), lambda i: (i, 0)),
            pl.BlockSpec((bm, kn), lambda i: (i, 0)),
            pl.BlockSpec((h, h), fixed),
            pl.BlockSpec((h, h), fixed),
        ],
        out_shape=[
            jax.ShapeDtypeStruct((n, kn), F32),
            jax.ShapeDtypeStruct((n, kn), F32),
            jax.ShapeDtypeStruct((h, h), F32),
            jax.ShapeDtypeStruct((h, h), F32),
        ],
        compiler_params=pltpu.CompilerParams(
            dimension_semantics=("arbitrary",)),
    )(x1, x2, w1c, w2c, wd1, w1sp, wd2, w2sp)


def _score(e, w, u_t):
    v = jnp.tanh(jnp.dot(e, w, preferred_element_type=F32))
    return jnp.sum(v * u_t, axis=1, keepdims=True)


def _att3(es, ef, ea, w, u_t):
    ss, sf, sa = _score(es, w, u_t), _score(ef, w, u_t), _score(ea, w, u_t)
    mx = jnp.maximum(jnp.maximum(ss, sf), sa)
    xs, xf, xa = jnp.exp(ss - mx), jnp.exp(sf - mx), jnp.exp(sa - mx)
    den = xs + xf + xa
    als, alf, ala = xs / den, xf / den, xa / den
    l = als * es + alf * ef + ala * ea
    return l, jnp.concatenate([als, alf, ala], axis=1)


def _agg_body(a1s_ref, a1f_ref, a1a_ref, a2s_ref, a2f_ref, a2a_ref,
              y1s_ref, y1f_ref, y1a_ref, y2s_ref, y2f_ref, y2a_ref,
              w1_ref, u1_ref, w2_ref, u2_ref, wc_ref, uc_ref,
              e1s_ref, e1f_ref, e1a_ref, e2s_ref, e2f_ref, e2a_ref,
              a1b_ref, a2b_ref, l1_ref, l2_ref, co_ref,
              al1_ref, al2_ref, alc_ref, *, nk):
    k = pl.program_id(1)

    @pl.when(k == 0)
    def _():
        for o in (e1s_ref, e1f_ref, e1a_ref, e2s_ref, e2f_ref, e2a_ref):
            o[...] = jnp.zeros_like(o)

    a1s = a1s_ref[...].astype(BF16)
    a2s = a2s_ref[...].astype(BF16)
    a1b_ref[...] = a1s
    a2b_ref[...] = a2s
    e1s_ref[...] += jnp.dot(a1s, y1s_ref[...].astype(BF16),
                            preferred_element_type=F32)
    e2s_ref[...] += jnp.dot(a2s, y2s_ref[...].astype(BF16),
                            preferred_element_type=F32)
    e1f_ref[...] += _dot(a1f_ref[...], y1f_ref[...])
    e1a_ref[...] += _dot(a1a_ref[...], y1a_ref[...])
    e2f_ref[...] += _dot(a2f_ref[...], y2f_ref[...])
    e2a_ref[...] += _dot(a2a_ref[...], y2a_ref[...])

    @pl.when(k == nk - 1)
    def _():
        l1, a1 = _att3(e1s_ref[...], e1f_ref[...], e1a_ref[...],
                       w1_ref[...], u1_ref[...])
        l2, a2 = _att3(e2s_ref[...], e2f_ref[...], e2a_ref[...],
                       w2_ref[...], u2_ref[...])
        s1 = _score(l1, wc_ref[...], uc_ref[...])
        s2 = _score(l2, wc_ref[...], uc_ref[...])
        mx = jnp.maximum(s1, s2)
        x1, x2 = jnp.exp(s1 - mx), jnp.exp(s2 - mx)
        den = x1 + x2
        b1, b2 = x1 / den, x2 / den
        l1_ref[...] = l1
        l2_ref[...] = l2
        co_ref[...] = b1 * l1 + b2 * l2
        al1_ref[...] = a1
        al2_ref[...] = a2
        alc_ref[...] = jnp.concatenate([b1, b2], axis=1)


def _agg(adjs, y1, y2, att_ws, att_us, h, bm, bk):
    import functools
    n = adjs[0].shape[0]
    nk = n // bk
    adj_spec = pl.BlockSpec((bm, bk), lambda i, k: (i, k))
    y_specs = [pl.BlockSpec((bk, h), lambda i, k, t=t: (k, t))
               for t in range(3)]
    fixed = lambda i, k: (0, 0)
    wb = pl.BlockSpec((h, h), fixed)
    ub = pl.BlockSpec((1, h), fixed)
    row_h = pl.BlockSpec((bm, h), lambda i, k: (i, 0))
    return pl.pallas_call(
        functools.partial(_agg_body, nk=nk),
        grid=(n // bm, nk),
        in_specs=([adj_spec] * 6 + y_specs + y_specs
                  + [wb, ub, wb, ub, wb, ub]),
        out_specs=[row_h] * 6 + [adj_spec, adj_spec]
                  + [row_h] * 3
                  + [pl.BlockSpec((bm, 3), lambda i, k: (i, 0)),
                     pl.BlockSpec((bm, 3), lambda i, k: (i, 0)),
                     pl.BlockSpec((bm, 2), lambda i, k: (i, 0))],
        out_shape=[jax.ShapeDtypeStruct((n, h), F32)] * 6
                  + [jax.ShapeDtypeStruct((n, n), BF16)] * 2
                  + [jax.ShapeDtypeStruct((n, h), F32)] * 3
                  + [jax.ShapeDtypeStruct((n, 3), F32),
                     jax.ShapeDtypeStruct((n, 3), F32),
                     jax.ShapeDtypeStruct((n, 2), F32)],
        compiler_params=pltpu.CompilerParams(
            dimension_semantics=("parallel", "arbitrary")),
    )(*adjs, y1, y1, y1, y2, y2, y2,
      att_ws[0], att_us[0].T, att_ws[1], att_us[1].T,
      att_ws[2], att_us[2].T)


def _u4_body(a1_ref, a2_ref, cb_ref, l1_ref, l2_ref, wd1_ref, wd2_ref,
             u1c_ref, u1l_ref, u2c_ref, u2l_ref, r1_ref, r2_ref, *, nk):
    k = pl.program_id(1)

    @pl.when(k == 0)
    def _():
        for o in (u1c_ref, u1l_ref, u2c_ref, u2l_ref):
            o[...] = jnp.zeros_like(o)

    a1 = a1_ref[...]
    a2 = a2_ref[...]
    cb = cb_ref[...].astype(BF16)
    u1c_ref[...] += jnp.dot(a1, cb, preferred_element_type=F32)
    u1l_ref[...] += jnp.dot(a1, l2_ref[...].astype(BF16),
                            preferred_element_type=F32)
    u2c_ref[...] += jnp.dot(a2, cb, preferred_element_type=F32)
    u2l_ref[...] += jnp.dot(a2, l1_ref[...].astype(BF16),
                            preferred_element_type=F32)

    @pl.when(k == nk - 1)
    def _():
        r1_ref[...] = _dot(u1c_ref[...], wd1_ref[...])
        r2_ref[...] = _dot(u2c_ref[...], wd2_ref[...])


def _u4(a1b, a2b, comb, l1, l2, wd1, wd2, bm, bk):
    import functools
    n, h = comb.shape
    d1 = wd1.shape[1]
    d2 = wd2.shape[1]
    nk = n // bk
    adj_spec = pl.BlockSpec((bm, bk), lambda i, k: (i, k))
    vec_spec = pl.BlockSpec((bk, h), lambda i, k: (k, 0))
    out_spec = pl.BlockSpec((bm, h), lambda i, k: (i, 0))
    fixed = lambda i, k: (0, 0)
    return pl.pallas_call(
        functools.partial(_u4_body, nk=nk),
        grid=(n // bm, nk),
        in_specs=[adj_spec, adj_spec, vec_spec, vec_spec, vec_spec,
                  pl.BlockSpec((h, d1), fixed), pl.BlockSpec((h, d2), fixed)],
        out_specs=[out_spec] * 4
                  + [pl.BlockSpec((bm, d1), lambda i, k: (i, 0)),
                     pl.BlockSpec((bm, d2), lambda i, k: (i, 0))],
        out_shape=[jax.ShapeDtypeStruct((n, h), F32)] * 4
                  + [jax.ShapeDtypeStruct((n, d1), F32),
                     jax.ShapeDtypeStruct((n, d2), F32)],
        compiler_params=pltpu.CompilerParams(
            dimension_semantics=("parallel", "arbitrary")),
    )(a1b, a2b, comb, l1, l2, wd1, wd2)


def _xr_body(a1_ref, a2_ref, u1l_ref, u2l_ref, md1_ref, md2_ref,
             x2_ref, x1_ref):
    @pl.when(pl.program_id(1) == 0)
    def _():
        x2_ref[...] = jnp.zeros_like(x2_ref)
        x1_ref[...] = jnp.zeros_like(x1_ref)

    z2 = _dot(u1l_ref[...], md1_ref[...])
    z1 = _dot(u2l_ref[...], md2_ref[...])
    x2_ref[...] += jnp.dot(a1_ref[...], z2.astype(BF16),
                           preferred_element_type=F32)
    x1_ref[...] += jnp.dot(a2_ref[...], z1.astype(BF16),
                           preferred_element_type=F32)


def _xr(a1b, a2b, u1l, u2l, md1, md2, bm, bk):
    n, h = u1l.shape
    adj_spec = pl.BlockSpec((bm, bk), lambda i, k: (i, k))
    vec_spec = pl.BlockSpec((bk, h), lambda i, k: (k, 0))
    md_spec = pl.BlockSpec((h, h), lambda i, k: (0, 0))
    out_spec = pl.BlockSpec((bm, h), lambda i, k: (i, 0))
    return pl.pallas_call(
        _xr_body,
        grid=(n // bm, n // bk),
        in_specs=[adj_spec, adj_spec, vec_spec, vec_spec, md_spec, md_spec],
        out_specs=[out_spec] * 2,
        out_shape=[jax.ShapeDtypeStruct((n, h), F32)] * 2,
        compiler_params=pltpu.CompilerParams(
            dimension_semantics=("parallel", "arbitrary")),
    )(a1b, a2b, u1l, u2l, md1, md2)


def kernel(features_omics1, features_omics2, adj_spatial_omics1,
           adj_feature_omics1, adj_augmented_omics1, adj_spatial_omics2,
           adj_feature_omics2, adj_augmented_omics2, W_enc1_sp, W_enc1_ft,
           W_enc1_aug, W_enc2_sp, W_enc2_ft, W_enc2_aug, W_dec1, W_dec2,
           att1_w, att1_u, att2_w, att2_u, attc_w, attc_u):
    n = features_omics1.shape[0]
    h = W_enc1_sp.shape[1]
    bm = min(512, n)
    bk = min(1024, n)

    w1c = jnp.concatenate([W_enc1_sp, W_enc1_ft, W_enc1_aug], axis=1)
    w2c = jnp.concatenate([W_enc2_sp, W_enc2_ft, W_enc2_aug], axis=1)
    y1, y2, md1, md2 = _proj(features_omics1, features_omics2, w1c, w2c,
                             W_dec1, W_enc1_sp, W_dec2, W_enc2_sp, bm=bm)

    (e1s, e1f, e1a, e2s, e2f, e2a, a1b, a2b,
     l1, l2, comb, al1, al2, alc) = _agg(
        (adj_spatial_omics1, adj_feature_omics1, adj_augmented_omics1,
         adj_spatial_omics2, adj_feature_omics2, adj_augmented_omics2),
        y1, y2, (att1_w, att2_w, attc_w), (att1_u, att2_u, attc_u),
        h, bm=bm, bk=bk)

    u1c, u1l, u2c, u2l, rec1, rec2 = _u4(
        a1b, a2b, comb, l1, l2, W_dec1, W_dec2, bm=bm, bk=bk)

    x2r, x1r = _xr(a1b, a2b, u1l, u2l, md1, md2, bm=bm, bk=bk)

    return (l1, l2, comb, rec1, rec2, x1r, x2r, al1, al2, alc,
            e1s, e1f, e1a, e2s, e2f, e2a)


# speedup vs baseline: 1.7231x; 1.1143x over previous
import functools

import jax
import jax.numpy as jnp
from jax.experimental import pallas as pl
from jax.experimental.pallas import tpu as pltpu

F32 = jnp.float32
BF16 = jnp.bfloat16


def _dot(x, y):
    return jnp.dot(x.astype(BF16), y.astype(BF16), preferred_element_type=F32)


def _proj_body(x1_ref, x2_ref, w1c_ref, w2c_ref, wd1_ref, w1sp_ref,
               wd2_ref, w2sp_ref, y1_ref, y2_ref, md1_ref, md2_ref):
    @pl.when(pl.program_id(0) == 0)
    def _():
        md1_ref[...] = jnp.dot(wd1_ref[...], w1sp_ref[...],
                               preferred_element_type=F32)
        md2_ref[...] = jnp.dot(wd2_ref[...], w2sp_ref[...],
                               preferred_element_type=F32)

    y1_ref[...] = _dot(x1_ref[...], w1c_ref[...])
    y2_ref[...] = _dot(x2_ref[...], w2c_ref[...])


def _proj(x1, x2, w1c, w2c, wd1, w1sp, wd2, w2sp, bm):
    n, d1 = x1.shape
    d2 = x2.shape[1]
    kn = w1c.shape[1]
    h = wd1.shape[0]
    fixed = lambda i: (0, 0)
    return pl.pallas_call(
        _proj_body,
        grid=(n // bm,),
        in_specs=[
            pl.BlockSpec((bm, d1), lambda i: (i, 0)),
            pl.BlockSpec((bm, d2), lambda i: (i, 0)),
            pl.BlockSpec((d1, kn), fixed),
            pl.BlockSpec((d2, kn), fixed),
            pl.BlockSpec((h, d1), fixed),
            pl.BlockSpec((d1, h), fixed),
            pl.BlockSpec((h, d2), fixed),
            pl.BlockSpec((d2, h), fixed),
        ],
        out_specs=[
            pl.BlockSpec((bm, kn), lambda i: (i, 0)),
            pl.BlockSpec((bm, kn), lambda i: (i, 0)),
            pl.BlockSpec((h, h), fixed),
            pl.BlockSpec((h, h), fixed),
        ],
        out_shape=[
            jax.ShapeDtypeStruct((n, kn), F32),
            jax.ShapeDtypeStruct((n, kn), F32),
            jax.ShapeDtypeStruct((h, h), F32),
            jax.ShapeDtypeStruct((h, h), F32),
        ],
        compiler_params=pltpu.CompilerParams(
            dimension_semantics=("arbitrary",)),
    )(x1, x2, w1c, w2c, wd1, w1sp, wd2, w2sp)


def _score(e, w, u_t):
    v = jnp.tanh(jnp.dot(e, w, preferred_element_type=F32))
    return jnp.sum(v * u_t, axis=1, keepdims=True)


def _att3(es, ef, ea, w, u_t):
    ss, sf, sa = _score(es, w, u_t), _score(ef, w, u_t), _score(ea, w, u_t)
    mx = jnp.maximum(jnp.maximum(ss, sf), sa)
    xs, xf, xa = jnp.exp(ss - mx), jnp.exp(sf - mx), jnp.exp(sa - mx)
    den = xs + xf + xa
    als, alf, ala = xs / den, xf / den, xa / den
    l = als * es + alf * ef + ala * ea
    return l, jnp.concatenate([als, alf, ala], axis=1)


def _agg_body(a1s_ref, a1f_ref, a1a_ref, a2s_ref, a2f_ref, a2a_ref,
              y1_ref, y2_ref,
              w1_ref, u1_ref, w2_ref, u2_ref, wc_ref, uc_ref,
              e1s_ref, e1f_ref, e1a_ref, e2s_ref, e2f_ref, e2a_ref,
              a1b_ref, a2b_ref, l1_ref, l2_ref, co_ref,
              al1_ref, al2_ref, alc_ref, *, nk, bk, h):
    k = pl.program_id(1)

    @pl.when(k == 0)
    def _():
        for o in (e1s_ref, e1f_ref, e1a_ref, e2s_ref, e2f_ref, e2a_ref):
            o[...] = jnp.zeros_like(o)

    rows = pl.ds(k * bk, bk)
    y1 = y1_ref[rows, :].astype(BF16)
    y2 = y2_ref[rows, :].astype(BF16)
    a1s = a1s_ref[...].astype(BF16)
    a2s = a2s_ref[...].astype(BF16)
    a1b_ref[...] = a1s
    a2b_ref[...] = a2s
    e1s_ref[...] += jnp.dot(a1s, y1[:, 0 * h:1 * h],
                            preferred_element_type=F32)
    e2s_ref[...] += jnp.dot(a2s, y2[:, 0 * h:1 * h],
                            preferred_element_type=F32)
    e1f_ref[...] += jnp.dot(a1f_ref[...].astype(BF16), y1[:, 1 * h:2 * h],
                            preferred_element_type=F32)
    e1a_ref[...] += jnp.dot(a1a_ref[...].astype(BF16), y1[:, 2 * h:3 * h],
                            preferred_element_type=F32)
    e2f_ref[...] += jnp.dot(a2f_ref[...].astype(BF16), y2[:, 1 * h:2 * h],
                            preferred_element_type=F32)
    e2a_ref[...] += jnp.dot(a2a_ref[...].astype(BF16), y2[:, 2 * h:3 * h],
                            preferred_element_type=F32)

    @pl.when(k == nk - 1)
    def _():
        l1, a1 = _att3(e1s_ref[...], e1f_ref[...], e1a_ref[...],
                       w1_ref[...], u1_ref[...])
        l2, a2 = _att3(e2s_ref[...], e2f_ref[...], e2a_ref[...],
                       w2_ref[...], u2_ref[...])
        s1 = _score(l1, wc_ref[...], uc_ref[...])
        s2 = _score(l2, wc_ref[...], uc_ref[...])
        mx = jnp.maximum(s1, s2)
        x1, x2 = jnp.exp(s1 - mx), jnp.exp(s2 - mx)
        den = x1 + x2
        b1, b2 = x1 / den, x2 / den
        l1_ref[...] = l1
        l2_ref[...] = l2
        co_ref[...] = b1 * l1 + b2 * l2
        al1_ref[...] = a1
        al2_ref[...] = a2
        alc_ref[...] = jnp.concatenate([b1, b2], axis=1)


def _agg(adjs, y1, y2, att_ws, att_us, h, bm, bk):
    n = adjs[0].shape[0]
    nk = n // bk
    kn = y1.shape[1]
    adj_spec = pl.BlockSpec((bm, bk), lambda i, k: (i, k))
    fixed = lambda i, k: (0, 0)
    y_spec = pl.BlockSpec((n, kn), fixed)
    wb = pl.BlockSpec((h, h), fixed)
    ub = pl.BlockSpec((1, h), fixed)
    row_h = pl.BlockSpec((bm, h), lambda i, k: (i, 0))
    return pl.pallas_call(
        functools.partial(_agg_body, nk=nk, bk=bk, h=h),
        grid=(n // bm, nk),
        in_specs=([adj_spec] * 6 + [y_spec, y_spec]
                  + [wb, ub, wb, ub, wb, ub]),
        out_specs=[row_h] * 6 + [adj_spec, adj_spec]
                  + [row_h] * 3
                  + [pl.BlockSpec((bm, 3), lambda i, k: (i, 0)),
                     pl.BlockSpec((bm, 3), lambda i, k: (i, 0)),
                     pl.BlockSpec((bm, 2), lambda i, k: (i, 0))],
        out_shape=[jax.ShapeDtypeStruct((n, h), F32)] * 6
                  + [jax.ShapeDtypeStruct((n, n), BF16)] * 2
                  + [jax.ShapeDtypeStruct((n, h), F32)] * 3
                  + [jax.ShapeDtypeStruct((n, 3), F32),
                     jax.ShapeDtypeStruct((n, 3), F32),
                     jax.ShapeDtypeStruct((n, 2), F32)],
        compiler_params=pltpu.CompilerParams(
            dimension_semantics=("parallel", "arbitrary")),
    )(*adjs, y1, y2,
      att_ws[0], att_us[0].T, att_ws[1], att_us[1].T,
      att_ws[2], att_us[2].T)


def _u4_body(a1_ref, a2_ref, c1_ref, c2_ref, wd1_ref, wd2_ref,
             u1_ref, u2_ref, r1_ref, r2_ref, *, nk, bk, h):
    k = pl.program_id(1)

    @pl.when(k == 0)
    def _():
        u1_ref[...] = jnp.zeros_like(u1_ref)
        u2_ref[...] = jnp.zeros_like(u2_ref)

    rows = pl.ds(k * bk, bk)
    u1_ref[...] += jnp.dot(a1_ref[...], c1_ref[rows, :].astype(BF16),
                           preferred_element_type=F32)
    u2_ref[...] += jnp.dot(a2_ref[...], c2_ref[rows, :].astype(BF16),
                           preferred_element_type=F32)

    @pl.when(k == nk - 1)
    def _():
        r1_ref[...] = _dot(u1_ref[:, 0:h], wd1_ref[...])
        r2_ref[...] = _dot(u2_ref[:, 0:h], wd2_ref[...])


def _u4(a1b, a2b, c1, c2, wd1, wd2, bm, bk):
    n = c1.shape[0]
    h2 = c1.shape[1]
    h = h2 // 2
    d1 = wd1.shape[1]
    d2 = wd2.shape[1]
    nk = n // bk
    adj_spec = pl.BlockSpec((bm, bk), lambda i, k: (i, k))
    fixed = lambda i, k: (0, 0)
    vec_spec = pl.BlockSpec((n, h2), fixed)
    out_spec = pl.BlockSpec((bm, h2), lambda i, k: (i, 0))
    return pl.pallas_call(
        functools.partial(_u4_body, nk=nk, bk=bk, h=h),
        grid=(n // bm, nk),
        in_specs=[adj_spec, adj_spec, vec_spec, vec_spec,
                  pl.BlockSpec((h, d1), fixed), pl.BlockSpec((h, d2), fixed)],
        out_specs=[out_spec, out_spec,
                   pl.BlockSpec((bm, d1), lambda i, k: (i, 0)),
                   pl.BlockSpec((bm, d2), lambda i, k: (i, 0))],
        out_shape=[jax.ShapeDtypeStruct((n, h2), F32),
                   jax.ShapeDtypeStruct((n, h2), F32),
                   jax.ShapeDtypeStruct((n, d1), F32),
                   jax.ShapeDtypeStruct((n, d2), F32)],
        compiler_params=pltpu.CompilerParams(
            dimension_semantics=("parallel", "arbitrary")),
    )(a1b, a2b, c1, c2, wd1, wd2)


def _xr_body(a1_ref, a2_ref, u1_ref, u2_ref, md1_ref, md2_ref,
             x2_ref, x1_ref, *, bk, h):
    k = pl.program_id(1)

    @pl.when(k == 0)
    def _():
        x2_ref[...] = jnp.zeros_like(x2_ref)
        x1_ref[...] = jnp.zeros_like(x1_ref)

    rows = pl.ds(k * bk, bk)
    z2 = _dot(u1_ref[rows, h:], md1_ref[...])
    z1 = _dot(u2_ref[rows, h:], md2_ref[...])
    x2_ref[...] += jnp.dot(a1_ref[...], z2.astype(BF16),
                           preferred_element_type=F32)
    x1_ref[...] += jnp.dot(a2_ref[...], z1.astype(BF16),
                           preferred_element_type=F32)


def _xr(a1b, a2b, u1, u2, md1, md2, bm, bk):
    n, h2 = u1.shape
    h = h2 // 2
    adj_spec = pl.BlockSpec((bm, bk), lambda i, k: (i, k))
    fixed = lambda i, k: (0, 0)
    vec_spec = pl.BlockSpec((n, h2), fixed)
    md_spec = pl.BlockSpec((h, h), fixed)
    out_spec = pl.BlockSpec((bm, h), lambda i, k: (i, 0))
    return pl.pallas_call(
        functools.partial(_xr_body, bk=bk, h=h),
        grid=(n // bm, n // bk),
        in_specs=[adj_spec, adj_spec, vec_spec, vec_spec, md_spec, md_spec],
        out_specs=[out_spec] * 2,
        out_shape=[jax.ShapeDtypeStruct((n, h), F32)] * 2,
        compiler_params=pltpu.CompilerParams(
            dimension_semantics=("parallel", "arbitrary")),
    )(a1b, a2b, u1, u2, md1, md2)


def kernel(features_omics1, features_omics2, adj_spatial_omics1,
           adj_feature_omics1, adj_augmented_omics1, adj_spatial_omics2,
           adj_feature_omics2, adj_augmented_omics2, W_enc1_sp, W_enc1_ft,
           W_enc1_aug, W_enc2_sp, W_enc2_ft, W_enc2_aug, W_dec1, W_dec2,
           att1_w, att1_u, att2_w, att2_u, attc_w, attc_u):
    n = features_omics1.shape[0]
    h = W_enc1_sp.shape[1]
    bm = min(512, n)
    bk = min(1024, n)
    bk2 = min(2048, n)

    w1c = jnp.concatenate([W_enc1_sp, W_enc1_ft, W_enc1_aug], axis=1)
    w2c = jnp.concatenate([W_enc2_sp, W_enc2_ft, W_enc2_aug], axis=1)
    y1, y2, md1, md2 = _proj(features_omics1, features_omics2, w1c, w2c,
                             W_dec1, W_enc1_sp, W_dec2, W_enc2_sp, bm=bm)

    (e1s, e1f, e1a, e2s, e2f, e2a, a1b, a2b,
     l1, l2, comb, al1, al2, alc) = _agg(
        (adj_spatial_omics1, adj_feature_omics1, adj_augmented_omics1,
         adj_spatial_omics2, adj_feature_omics2, adj_augmented_omics2),
        y1, y2, (att1_w, att2_w, attc_w), (att1_u, att2_u, attc_u),
        h, bm=bm, bk=bk)

    c1 = jnp.concatenate([comb, l2], axis=1)
    c2 = jnp.concatenate([comb, l1], axis=1)
    u1, u2, rec1, rec2 = _u4(a1b, a2b, c1, c2, W_dec1, W_dec2, bm=bm, bk=bk2)

    x2r, x1r = _xr(a1b, a2b, u1, u2, md1, md2, bm=bm, bk=bk2)

    return (l1, l2, comb, rec1, rec2, x1r, x2r, al1, al2, alc,
            e1s, e1f, e1a, e2s, e2f, e2a)


# speedup vs baseline: 1.8094x; 1.0501x over previous
import functools

import jax
import jax.numpy as jnp
from jax.experimental import pallas as pl
from jax.experimental.pallas import tpu as pltpu

F32 = jnp.float32
BF16 = jnp.bfloat16


def _dot(x, y):
    return jnp.dot(x.astype(BF16), y.astype(BF16), preferred_element_type=F32)


def _proj_body(x1_ref, x2_ref, w1c_ref, w2c_ref, wd1_ref, w1sp_ref,
               wd2_ref, w2sp_ref, y1_ref, y2_ref, md1_ref, md2_ref):
    @pl.when(pl.program_id(0) == 0)
    def _():
        md1_ref[...] = jnp.dot(wd1_ref[...], w1sp_ref[...],
                               preferred_element_type=F32)
        md2_ref[...] = jnp.dot(wd2_ref[...], w2sp_ref[...],
                               preferred_element_type=F32)

    y1_ref[...] = _dot(x1_ref[...], w1c_ref[...])
    y2_ref[...] = _dot(x2_ref[...], w2c_ref[...])


def _proj(x1, x2, w1c, w2c, wd1, w1sp, wd2, w2sp, bm):
    n, d1 = x1.shape
    d2 = x2.shape[1]
    kn = w1c.shape[1]
    h = wd1.shape[0]
    fixed = lambda i: (0, 0)
    return pl.pallas_call(
        _proj_body,
        grid=(n // bm,),
        in_specs=[
            pl.BlockSpec((bm, d1), lambda i: (i, 0)),
            pl.BlockSpec((bm, d2), lambda i: (i, 0)),
            pl.BlockSpec((d1, kn), fixed),
            pl.BlockSpec((d2, kn), fixed),
            pl.BlockSpec((h, d1), fixed),
            pl.BlockSpec((d1, h), fixed),
            pl.BlockSpec((h, d2), fixed),
            pl.BlockSpec((d2, h), fixed),
        ],
        out_specs=[
            pl.BlockSpec((bm, kn), lambda i: (i, 0)),
            pl.BlockSpec((bm, kn), lambda i: (i, 0)),
            pl.BlockSpec((h, h), fixed),
            pl.BlockSpec((h, h), fixed),
        ],
        out_shape=[
            jax.ShapeDtypeStruct((n, kn), F32),
            jax.ShapeDtypeStruct((n, kn), F32),
            jax.ShapeDtypeStruct((h, h), F32),
            jax.ShapeDtypeStruct((h, h), F32),
        ],
        compiler_params=pltpu.CompilerParams(
            dimension_semantics=("arbitrary",)),
    )(x1, x2, w1c, w2c, wd1, w1sp, wd2, w2sp)


def _score(e, w, u_t):
    v = jnp.tanh(jnp.dot(e, w, preferred_element_type=F32))
    return jnp.sum(v * u_t, axis=1, keepdims=True)


def _att3(es, ef, ea, w, u_t):
    ss, sf, sa = _score(es, w, u_t), _score(ef, w, u_t), _score(ea, w, u_t)
    mx = jnp.maximum(jnp.maximum(ss, sf), sa)
    xs, xf, xa = jnp.exp(ss - mx), jnp.exp(sf - mx), jnp.exp(sa - mx)
    den = xs + xf + xa
    als, alf, ala = xs / den, xf / den, xa / den
    l = als * es + alf * ef + ala * ea
    return l, jnp.concatenate([als, alf, ala], axis=1)


def _agg_body(a1s_ref, a1f_ref, a1a_ref, a2s_ref, a2f_ref, a2a_ref,
              y1_ref, y2_ref,
              w1_ref, u1_ref, w2_ref, u2_ref, wc_ref, uc_ref,
              e1s_ref, e1f_ref, e1a_ref, e2s_ref, e2f_ref, e2a_ref,
              a1b_ref, a2b_ref, l1_ref, l2_ref, co_ref,
              al1_ref, al2_ref, alc_ref, *, nk, bk, h):
    k = pl.program_id(1)

    @pl.when(k == 0)
    def _():
        for o in (e1s_ref, e1f_ref, e1a_ref, e2s_ref, e2f_ref, e2a_ref):
            o[...] = jnp.zeros_like(o)

    rows = pl.ds(k * bk, bk)
    y1 = y1_ref[rows, :].astype(BF16)
    y2 = y2_ref[rows, :].astype(BF16)
    a1s = a1s_ref[...].astype(BF16)
    a2s = a2s_ref[...].astype(BF16)
    a1b_ref[...] = a1s
    a2b_ref[...] = a2s
    e1s_ref[...] += jnp.dot(a1s, y1[:, 0 * h:1 * h],
                            preferred_element_type=F32)
    e2s_ref[...] += jnp.dot(a2s, y2[:, 0 * h:1 * h],
                            preferred_element_type=F32)
    e1f_ref[...] += jnp.dot(a1f_ref[...].astype(BF16), y1[:, 1 * h:2 * h],
                            preferred_element_type=F32)
    e1a_ref[...] += jnp.dot(a1a_ref[...].astype(BF16), y1[:, 2 * h:3 * h],
                            preferred_element_type=F32)
    e2f_ref[...] += jnp.dot(a2f_ref[...].astype(BF16), y2[:, 1 * h:2 * h],
                            preferred_element_type=F32)
    e2a_ref[...] += jnp.dot(a2a_ref[...].astype(BF16), y2[:, 2 * h:3 * h],
                            preferred_element_type=F32)

    @pl.when(k == nk - 1)
    def _():
        l1, a1 = _att3(e1s_ref[...], e1f_ref[...], e1a_ref[...],
                       w1_ref[...], u1_ref[...])
        l2, a2 = _att3(e2s_ref[...], e2f_ref[...], e2a_ref[...],
                       w2_ref[...], u2_ref[...])
        s1 = _score(l1, wc_ref[...], uc_ref[...])
        s2 = _score(l2, wc_ref[...], uc_ref[...])
        mx = jnp.maximum(s1, s2)
        x1, x2 = jnp.exp(s1 - mx), jnp.exp(s2 - mx)
        den = x1 + x2
        b1, b2 = x1 / den, x2 / den
        l1_ref[...] = l1
        l2_ref[...] = l2
        co_ref[...] = b1 * l1 + b2 * l2
        al1_ref[...] = a1
        al2_ref[...] = a2
        alc_ref[...] = jnp.concatenate([b1, b2], axis=1)


def _agg(adjs, y1, y2, att_ws, att_us, h, bm, bk):
    n = adjs[0].shape[0]
    nk = n // bk
    kn = y1.shape[1]
    adj_spec = pl.BlockSpec((bm, bk), lambda i, k: (i, k))
    fixed = lambda i, k: (0, 0)
    y_spec = pl.BlockSpec((n, kn), fixed)
    wb = pl.BlockSpec((h, h), fixed)
    ub = pl.BlockSpec((1, h), fixed)
    row_h = pl.BlockSpec((bm, h), lambda i, k: (i, 0))
    return pl.pallas_call(
        functools.partial(_agg_body, nk=nk, bk=bk, h=h),
        grid=(n // bm, nk),
        in_specs=([adj_spec] * 6 + [y_spec, y_spec]
                  + [wb, ub, wb, ub, wb, ub]),
        out_specs=[row_h] * 6 + [adj_spec, adj_spec]
                  + [row_h] * 3
                  + [pl.BlockSpec((bm, 3), lambda i, k: (i, 0)),
                     pl.BlockSpec((bm, 3), lambda i, k: (i, 0)),
                     pl.BlockSpec((bm, 2), lambda i, k: (i, 0))],
        out_shape=[jax.ShapeDtypeStruct((n, h), F32)] * 6
                  + [jax.ShapeDtypeStruct((n, n), BF16)] * 2
                  + [jax.ShapeDtypeStruct((n, h), F32)] * 3
                  + [jax.ShapeDtypeStruct((n, 3), F32),
                     jax.ShapeDtypeStruct((n, 3), F32),
                     jax.ShapeDtypeStruct((n, 2), F32)],
        compiler_params=pltpu.CompilerParams(
            dimension_semantics=("parallel", "arbitrary")),
    )(*adjs, y1, y2,
      att_ws[0], att_us[0].T, att_ws[1], att_us[1].T,
      att_ws[2], att_us[2].T)


def _u4_body(a1_ref, a2_ref, c1_ref, c2_ref, wd1_ref, wd2_ref,
             u1_ref, u2_ref, r1_ref, r2_ref, *, nk, bk, h):
    k = pl.program_id(1)

    @pl.when(k == 0)
    def _():
        u1_ref[...] = jnp.zeros_like(u1_ref)
        u2_ref[...] = jnp.zeros_like(u2_ref)

    rows = pl.ds(k * bk, bk)
    u1_ref[...] += jnp.dot(a1_ref[...], c1_ref[rows, :].astype(BF16),
                           preferred_element_type=F32)
    u2_ref[...] += jnp.dot(a2_ref[...], c2_ref[rows, :].astype(BF16),
                           preferred_element_type=F32)

    @pl.when(k == nk - 1)
    def _():
        r1_ref[...] = _dot(u1_ref[:, 0:h], wd1_ref[...])
        r2_ref[...] = _dot(u2_ref[:, 0:h], wd2_ref[...])


def _u4(a1b, a2b, c1, c2, wd1, wd2, bm, bk):
    n = c1.shape[0]
    h2 = c1.shape[1]
    h = h2 // 2
    d1 = wd1.shape[1]
    d2 = wd2.shape[1]
    nk = n // bk
    adj_spec = pl.BlockSpec((bm, bk), lambda i, k: (i, k))
    fixed = lambda i, k: (0, 0)
    vec_spec = pl.BlockSpec((n, h2), fixed)
    out_spec = pl.BlockSpec((bm, h2), lambda i, k: (i, 0))
    return pl.pallas_call(
        functools.partial(_u4_body, nk=nk, bk=bk, h=h),
        grid=(n // bm, nk),
        in_specs=[adj_spec, adj_spec, vec_spec, vec_spec,
                  pl.BlockSpec((h, d1), fixed), pl.BlockSpec((h, d2), fixed)],
        out_specs=[out_spec, out_spec,
                   pl.BlockSpec((bm, d1), lambda i, k: (i, 0)),
                   pl.BlockSpec((bm, d2), lambda i, k: (i, 0))],
        out_shape=[jax.ShapeDtypeStruct((n, h2), F32),
                   jax.ShapeDtypeStruct((n, h2), F32),
                   jax.ShapeDtypeStruct((n, d1), F32),
                   jax.ShapeDtypeStruct((n, d2), F32)],
        compiler_params=pltpu.CompilerParams(
            dimension_semantics=("parallel", "arbitrary")),
    )(a1b, a2b, c1, c2, wd1, wd2)


def _xr_body(a1_ref, a2_ref, u1_ref, u2_ref, md1_ref, md2_ref,
             x2_ref, x1_ref, *, bk, h):
    k = pl.program_id(1)

    @pl.when(k == 0)
    def _():
        x2_ref[...] = jnp.zeros_like(x2_ref)
        x1_ref[...] = jnp.zeros_like(x1_ref)

    rows = pl.ds(k * bk, bk)
    z2 = _dot(u1_ref[rows, h:], md1_ref[...])
    z1 = _dot(u2_ref[rows, h:], md2_ref[...])
    x2_ref[...] += jnp.dot(a1_ref[...], z2.astype(BF16),
                           preferred_element_type=F32)
    x1_ref[...] += jnp.dot(a2_ref[...], z1.astype(BF16),
                           preferred_element_type=F32)


def _xr(a1b, a2b, u1, u2, md1, md2, bm, bk):
    n, h2 = u1.shape
    h = h2 // 2
    adj_spec = pl.BlockSpec((bm, bk), lambda i, k: (i, k))
    fixed = lambda i, k: (0, 0)
    vec_spec = pl.BlockSpec((n, h2), fixed)
    md_spec = pl.BlockSpec((h, h), fixed)
    out_spec = pl.BlockSpec((bm, h), lambda i, k: (i, 0))
    return pl.pallas_call(
        functools.partial(_xr_body, bk=bk, h=h),
        grid=(n // bm, n // bk),
        in_specs=[adj_spec, adj_spec, vec_spec, vec_spec, md_spec, md_spec],
        out_specs=[out_spec] * 2,
        out_shape=[jax.ShapeDtypeStruct((n, h), F32)] * 2,
        compiler_params=pltpu.CompilerParams(
            dimension_semantics=("parallel", "arbitrary")),
    )(a1b, a2b, u1, u2, md1, md2)


def kernel(features_omics1, features_omics2, adj_spatial_omics1,
           adj_feature_omics1, adj_augmented_omics1, adj_spatial_omics2,
           adj_feature_omics2, adj_augmented_omics2, W_enc1_sp, W_enc1_ft,
           W_enc1_aug, W_enc2_sp, W_enc2_ft, W_enc2_aug, W_dec1, W_dec2,
           att1_w, att1_u, att2_w, att2_u, attc_w, attc_u):
    n = features_omics1.shape[0]
    h = W_enc1_sp.shape[1]
    bm = min(512, n)
    bk = min(1024, n)
    bk2 = min(4096, n)

    w1c = jnp.concatenate([W_enc1_sp, W_enc1_ft, W_enc1_aug], axis=1)
    w2c = jnp.concatenate([W_enc2_sp, W_enc2_ft, W_enc2_aug], axis=1)
    y1, y2, md1, md2 = _proj(features_omics1, features_omics2, w1c, w2c,
                             W_dec1, W_enc1_sp, W_dec2, W_enc2_sp, bm=bm)

    (e1s, e1f, e1a, e2s, e2f, e2a, a1b, a2b,
     l1, l2, comb, al1, al2, alc) = _agg(
        (adj_spatial_omics1, adj_feature_omics1, adj_augmented_omics1,
         adj_spatial_omics2, adj_feature_omics2, adj_augmented_omics2),
        y1, y2, (att1_w, att2_w, attc_w), (att1_u, att2_u, attc_u),
        h, bm=bm, bk=bk)

    c1 = jnp.concatenate([comb, l2], axis=1)
    c2 = jnp.concatenate([comb, l1], axis=1)
    u1, u2, rec1, rec2 = _u4(a1b, a2b, c1, c2, W_dec1, W_dec2, bm=bm, bk=bk2)

    x2r, x1r = _xr(a1b, a2b, u1, u2, md1, md2, bm=bm, bk=bk2)

    return (l1, l2, comb, rec1, rec2, x1r, x2r, al1, al2, alc,
            e1s, e1f, e1a, e2s, e2f, e2a)


# speedup vs baseline: 1.8111x; 1.0009x over previous
import functools

import jax
import jax.numpy as jnp
from jax.experimental import pallas as pl
from jax.experimental.pallas import tpu as pltpu

F32 = jnp.float32
BF16 = jnp.bfloat16


def _dot(x, y):
    return jnp.dot(x.astype(BF16), y.astype(BF16), preferred_element_type=F32)


def _proj_body(x1_ref, x2_ref, w1c_ref, w2c_ref, wd1_ref, w1sp_ref,
               wd2_ref, w2sp_ref, y1_ref, y2_ref, md1_ref, md2_ref):
    @pl.when(pl.program_id(0) == 0)
    def _():
        md1_ref[...] = jnp.dot(wd1_ref[...], w1sp_ref[...],
                               preferred_element_type=F32)
        md2_ref[...] = jnp.dot(wd2_ref[...], w2sp_ref[...],
                               preferred_element_type=F32)

    y1_ref[...] = _dot(x1_ref[...], w1c_ref[...])
    y2_ref[...] = _dot(x2_ref[...], w2c_ref[...])


def _proj(x1, x2, w1c, w2c, wd1, w1sp, wd2, w2sp, bm):
    n, d1 = x1.shape
    d2 = x2.shape[1]
    kn = w1c.shape[1]
    h = wd1.shape[0]
    fixed = lambda i: (0, 0)
    return pl.pallas_call(
        _proj_body,
        grid=(n // bm,),
        in_specs=[
            pl.BlockSpec((bm, d1), lambda i: (i, 0)),
            pl.BlockSpec((bm, d2), lambda i: (i, 0)),
            pl.BlockSpec((d1, kn), fixed),
            pl.BlockSpec((d2, kn), fixed),
            pl.BlockSpec((h, d1), fixed),
            pl.BlockSpec((d1, h), fixed),
            pl.BlockSpec((h, d2), fixed),
            pl.BlockSpec((d2, h), fixed),
        ],
        out_specs=[
            pl.BlockSpec((bm, kn), lambda i: (i, 0)),
            pl.BlockSpec((bm, kn), lambda i: (i, 0)),
            pl.BlockSpec((h, h), fixed),
            pl.BlockSpec((h, h), fixed),
        ],
        out_shape=[
            jax.ShapeDtypeStruct((n, kn), F32),
            jax.ShapeDtypeStruct((n, kn), F32),
            jax.ShapeDtypeStruct((h, h), F32),
            jax.ShapeDtypeStruct((h, h), F32),
        ],
        compiler_params=pltpu.CompilerParams(
            dimension_semantics=("arbitrary",)),
    )(x1, x2, w1c, w2c, wd1, w1sp, wd2, w2sp)


def _score(e, w, u_t):
    v = jnp.tanh(jnp.dot(e, w, preferred_element_type=F32))
    return jnp.sum(v * u_t, axis=1, keepdims=True)


def _att3(es, ef, ea, w, u_t):
    ss, sf, sa = _score(es, w, u_t), _score(ef, w, u_t), _score(ea, w, u_t)
    mx = jnp.maximum(jnp.maximum(ss, sf), sa)
    xs, xf, xa = jnp.exp(ss - mx), jnp.exp(sf - mx), jnp.exp(sa - mx)
    den = xs + xf + xa
    als, alf, ala = xs / den, xf / den, xa / den
    l = als * es + alf * ef + ala * ea
    return l, jnp.concatenate([als, alf, ala], axis=1)


def _agg_body(a1s_ref, a1f_ref, a1a_ref, a2s_ref, a2f_ref, a2a_ref,
              y1_ref, y2_ref,
              w1_ref, u1_ref, w2_ref, u2_ref, wc_ref, uc_ref,
              e1s_ref, e1f_ref, e1a_ref, e2s_ref, e2f_ref, e2a_ref,
              a1b_ref, a2b_ref, l1_ref, l2_ref, co_ref,
              al1_ref, al2_ref, alc_ref, *, nk, bk, h):
    k = pl.program_id(1)

    @pl.when(k == 0)
    def _():
        for o in (e1s_ref, e1f_ref, e1a_ref, e2s_ref, e2f_ref, e2a_ref):
            o[...] = jnp.zeros_like(o)

    rows = pl.ds(k * bk, bk)
    y1 = y1_ref[rows, :].astype(BF16)
    y2 = y2_ref[rows, :].astype(BF16)
    a1s = a1s_ref[...].astype(BF16)
    a2s = a2s_ref[...].astype(BF16)
    a1b_ref[...] = a1s
    a2b_ref[...] = a2s
    e1s_ref[...] += jnp.dot(a1s, y1[:, 0 * h:1 * h],
                            preferred_element_type=F32)
    e2s_ref[...] += jnp.dot(a2s, y2[:, 0 * h:1 * h],
                            preferred_element_type=F32)
    e1f_ref[...] += jnp.dot(a1f_ref[...].astype(BF16), y1[:, 1 * h:2 * h],
                            preferred_element_type=F32)
    e1a_ref[...] += jnp.dot(a1a_ref[...].astype(BF16), y1[:, 2 * h:3 * h],
                            preferred_element_type=F32)
    e2f_ref[...] += jnp.dot(a2f_ref[...].astype(BF16), y2[:, 1 * h:2 * h],
                            preferred_element_type=F32)
    e2a_ref[...] += jnp.dot(a2a_ref[...].astype(BF16), y2[:, 2 * h:3 * h],
                            preferred_element_type=F32)

    @pl.when(k == nk - 1)
    def _():
        l1, a1 = _att3(e1s_ref[...], e1f_ref[...], e1a_ref[...],
                       w1_ref[...], u1_ref[...])
        l2, a2 = _att3(e2s_ref[...], e2f_ref[...], e2a_ref[...],
                       w2_ref[...], u2_ref[...])
        s1 = _score(l1, wc_ref[...], uc_ref[...])
        s2 = _score(l2, wc_ref[...], uc_ref[...])
        mx = jnp.maximum(s1, s2)
        x1, x2 = jnp.exp(s1 - mx), jnp.exp(s2 - mx)
        den = x1 + x2
        b1, b2 = x1 / den, x2 / den
        l1_ref[...] = l1
        l2_ref[...] = l2
        co_ref[...] = b1 * l1 + b2 * l2
        al1_ref[...] = a1
        al2_ref[...] = a2
        alc_ref[...] = jnp.concatenate([b1, b2], axis=1)


def _agg(adjs, y1, y2, att_ws, att_us, h, bm, bk):
    n = adjs[0].shape[0]
    nk = n // bk
    kn = y1.shape[1]
    adj_spec = pl.BlockSpec((bm, bk), lambda i, k: (i, k))
    fixed = lambda i, k: (0, 0)
    y_spec = pl.BlockSpec((n, kn), fixed)
    wb = pl.BlockSpec((h, h), fixed)
    ub = pl.BlockSpec((1, h), fixed)
    row_h = pl.BlockSpec((bm, h), lambda i, k: (i, 0))
    return pl.pallas_call(
        functools.partial(_agg_body, nk=nk, bk=bk, h=h),
        grid=(n // bm, nk),
        in_specs=([adj_spec] * 6 + [y_spec, y_spec]
                  + [wb, ub, wb, ub, wb, ub]),
        out_specs=[row_h] * 6 + [adj_spec, adj_spec]
                  + [row_h] * 3
                  + [pl.BlockSpec((bm, 3), lambda i, k: (i, 0)),
                     pl.BlockSpec((bm, 3), lambda i, k: (i, 0)),
                     pl.BlockSpec((bm, 2), lambda i, k: (i, 0))],
        out_shape=[jax.ShapeDtypeStruct((n, h), F32)] * 6
                  + [jax.ShapeDtypeStruct((n, n), BF16)] * 2
                  + [jax.ShapeDtypeStruct((n, h), F32)] * 3
                  + [jax.ShapeDtypeStruct((n, 3), F32),
                     jax.ShapeDtypeStruct((n, 3), F32),
                     jax.ShapeDtypeStruct((n, 2), F32)],
        compiler_params=pltpu.CompilerParams(
            dimension_semantics=("parallel", "arbitrary")),
    )(*adjs, y1, y2,
      att_ws[0], att_us[0].T, att_ws[1], att_us[1].T,
      att_ws[2], att_us[2].T)


def _u4_body(a1_ref, a2_ref, c1_ref, c2_ref, wd1_ref, wd2_ref,
             u1_ref, u2_ref, r1_ref, r2_ref, *, nk, bk, h):
    k = pl.program_id(1)

    @pl.when(k == 0)
    def _():
        u1_ref[...] = jnp.zeros_like(u1_ref)
        u2_ref[...] = jnp.zeros_like(u2_ref)

    rows = pl.ds(k * bk, bk)
    u1_ref[...] += jnp.dot(a1_ref[...], c1_ref[rows, :].astype(BF16),
                           preferred_element_type=F32)
    u2_ref[...] += jnp.dot(a2_ref[...], c2_ref[rows, :].astype(BF16),
                           preferred_element_type=F32)

    @pl.when(k == nk - 1)
    def _():
        r1_ref[...] = _dot(u1_ref[:, 0:h], wd1_ref[...])
        r2_ref[...] = _dot(u2_ref[:, 0:h], wd2_ref[...])


def _u4(a1b, a2b, c1, c2, wd1, wd2, bm, bk):
    n = c1.shape[0]
    h2 = c1.shape[1]
    h = h2 // 2
    d1 = wd1.shape[1]
    d2 = wd2.shape[1]
    nk = n // bk
    adj_spec = pl.BlockSpec((bm, bk), lambda i, k: (i, k))
    fixed = lambda i, k: (0, 0)
    vec_spec = pl.BlockSpec((n, h2), fixed)
    out_spec = pl.BlockSpec((bm, h2), lambda i, k: (i, 0))
    return pl.pallas_call(
        functools.partial(_u4_body, nk=nk, bk=bk, h=h),
        grid=(n // bm, nk),
        in_specs=[adj_spec, adj_spec, vec_spec, vec_spec,
                  pl.BlockSpec((h, d1), fixed), pl.BlockSpec((h, d2), fixed)],
        out_specs=[out_spec, out_spec,
                   pl.BlockSpec((bm, d1), lambda i, k: (i, 0)),
                   pl.BlockSpec((bm, d2), lambda i, k: (i, 0))],
        out_shape=[jax.ShapeDtypeStruct((n, h2), F32),
                   jax.ShapeDtypeStruct((n, h2), F32),
                   jax.ShapeDtypeStruct((n, d1), F32),
                   jax.ShapeDtypeStruct((n, d2), F32)],
        compiler_params=pltpu.CompilerParams(
            dimension_semantics=("parallel", "arbitrary")),
    )(a1b, a2b, c1, c2, wd1, wd2)


def _xr_body(a1_ref, a2_ref, u1_ref, u2_ref, md1_ref, md2_ref,
             x2_ref, x1_ref, *, bk, h):
    k = pl.program_id(1)

    @pl.when(k == 0)
    def _():
        x2_ref[...] = jnp.zeros_like(x2_ref)
        x1_ref[...] = jnp.zeros_like(x1_ref)

    rows = pl.ds(k * bk, bk)
    z2 = _dot(u1_ref[rows, h:], md1_ref[...])
    z1 = _dot(u2_ref[rows, h:], md2_ref[...])
    x2_ref[...] += jnp.dot(a1_ref[...], z2.astype(BF16),
                           preferred_element_type=F32)
    x1_ref[...] += jnp.dot(a2_ref[...], z1.astype(BF16),
                           preferred_element_type=F32)


def _xr(a1b, a2b, u1, u2, md1, md2, bm, bk):
    n, h2 = u1.shape
    h = h2 // 2
    adj_spec = pl.BlockSpec((bm, bk), lambda i, k: (i, k))
    fixed = lambda i, k: (0, 0)
    vec_spec = pl.BlockSpec((n, h2), fixed)
    md_spec = pl.BlockSpec((h, h), fixed)
    out_spec = pl.BlockSpec((bm, h), lambda i, k: (i, 0))
    return pl.pallas_call(
        functools.partial(_xr_body, bk=bk, h=h),
        grid=(n // bm, n // bk),
        in_specs=[adj_spec, adj_spec, vec_spec, vec_spec, md_spec, md_spec],
        out_specs=[out_spec] * 2,
        out_shape=[jax.ShapeDtypeStruct((n, h), F32)] * 2,
        compiler_params=pltpu.CompilerParams(
            dimension_semantics=("parallel", "arbitrary")),
    )(a1b, a2b, u1, u2, md1, md2)


def kernel(features_omics1, features_omics2, adj_spatial_omics1,
           adj_feature_omics1, adj_augmented_omics1, adj_spatial_omics2,
           adj_feature_omics2, adj_augmented_omics2, W_enc1_sp, W_enc1_ft,
           W_enc1_aug, W_enc2_sp, W_enc2_ft, W_enc2_aug, W_dec1, W_dec2,
           att1_w, att1_u, att2_w, att2_u, attc_w, attc_u):
    n = features_omics1.shape[0]
    h = W_enc1_sp.shape[1]
    bm = min(512, n)
    bma = min(256, n)
    bk = min(2048, n)
    bk2 = min(4096, n)

    w1c = jnp.concatenate([W_enc1_sp, W_enc1_ft, W_enc1_aug], axis=1)
    w2c = jnp.concatenate([W_enc2_sp, W_enc2_ft, W_enc2_aug], axis=1)
    y1, y2, md1, md2 = _proj(features_omics1, features_omics2, w1c, w2c,
                             W_dec1, W_enc1_sp, W_dec2, W_enc2_sp, bm=bm)

    (e1s, e1f, e1a, e2s, e2f, e2a, a1b, a2b,
     l1, l2, comb, al1, al2, alc) = _agg(
        (adj_spatial_omics1, adj_feature_omics1, adj_augmented_omics1,
         adj_spatial_omics2, adj_feature_omics2, adj_augmented_omics2),
        y1, y2, (att1_w, att2_w, attc_w), (att1_u, att2_u, attc_u),
        h, bm=bma, bk=bk)

    c1 = jnp.concatenate([comb, l2], axis=1)
    c2 = jnp.concatenate([comb, l1], axis=1)
    u1, u2, rec1, rec2 = _u4(a1b, a2b, c1, c2, W_dec1, W_dec2, bm=bm, bk=bk2)

    x2r, x1r = _xr(a1b, a2b, u1, u2, md1, md2, bm=bm, bk=bk2)

    return (l1, l2, comb, rec1, rec2, x1r, x2r, al1, al2, alc,
            e1s, e1f, e1a, e2s, e2f, e2a)


# speedup vs baseline: 1.8315x; 1.0113x over previous
import functools

import jax
import jax.numpy as jnp
from jax.experimental import pallas as pl
from jax.experimental.pallas import tpu as pltpu

F32 = jnp.float32
BF16 = jnp.bfloat16


def _dot(x, y):
    return jnp.dot(x.astype(BF16), y.astype(BF16), preferred_element_type=F32)


def _proj_body(x1_ref, x2_ref, w1c_ref, w2c_ref, wd1_ref, w1sp_ref,
               wd2_ref, w2sp_ref, y1_ref, y2_ref, md1_ref, md2_ref):
    @pl.when(pl.program_id(0) == 0)
    def _():
        md1_ref[...] = jnp.dot(wd1_ref[...], w1sp_ref[...],
                               preferred_element_type=F32)
        md2_ref[...] = jnp.dot(wd2_ref[...], w2sp_ref[...],
                               preferred_element_type=F32)

    y1_ref[...] = _dot(x1_ref[...], w1c_ref[...])
    y2_ref[...] = _dot(x2_ref[...], w2c_ref[...])


def _proj(x1, x2, w1c, w2c, wd1, w1sp, wd2, w2sp, bm):
    n, d1 = x1.shape
    d2 = x2.shape[1]
    kn = w1c.shape[1]
    h = wd1.shape[0]
    fixed = lambda i: (0, 0)
    return pl.pallas_call(
        _proj_body,
        grid=(n // bm,),
        in_specs=[
            pl.BlockSpec((bm, d1), lambda i: (i, 0)),
            pl.BlockSpec((bm, d2), lambda i: (i, 0)),
            pl.BlockSpec((d1, kn), fixed),
            pl.BlockSpec((d2, kn), fixed),
            pl.BlockSpec((h, d1), fixed),
            pl.BlockSpec((d1, h), fixed),
            pl.BlockSpec((h, d2), fixed),
            pl.BlockSpec((d2, h), fixed),
        ],
        out_specs=[
            pl.BlockSpec((bm, kn), lambda i: (i, 0)),
            pl.BlockSpec((bm, kn), lambda i: (i, 0)),
            pl.BlockSpec((h, h), fixed),
            pl.BlockSpec((h, h), fixed),
        ],
        out_shape=[
            jax.ShapeDtypeStruct((n, kn), F32),
            jax.ShapeDtypeStruct((n, kn), F32),
            jax.ShapeDtypeStruct((h, h), F32),
            jax.ShapeDtypeStruct((h, h), F32),
        ],
        compiler_params=pltpu.CompilerParams(
            dimension_semantics=("arbitrary",)),
    )(x1, x2, w1c, w2c, wd1, w1sp, wd2, w2sp)


def _score(e, w, u_t):
    v = jnp.tanh(jnp.dot(e, w, preferred_element_type=F32))
    return jnp.sum(v * u_t, axis=1, keepdims=True)


def _att3(es, ef, ea, w, u_t):
    ss, sf, sa = _score(es, w, u_t), _score(ef, w, u_t), _score(ea, w, u_t)
    mx = jnp.maximum(jnp.maximum(ss, sf), sa)
    xs, xf, xa = jnp.exp(ss - mx), jnp.exp(sf - mx), jnp.exp(sa - mx)
    den = xs + xf + xa
    als, alf, ala = xs / den, xf / den, xa / den
    l = als * es + alf * ef + ala * ea
    return l, jnp.concatenate([als, alf, ala], axis=1)


def _agg_body(a1s_ref, a1f_ref, a1a_ref, a2s_ref, a2f_ref, a2a_ref,
              y1_ref, y2_ref,
              w1_ref, u1_ref, w2_ref, u2_ref, wc_ref, uc_ref,
              e1s_ref, e1f_ref, e1a_ref, e2s_ref, e2f_ref, e2a_ref,
              a1b_ref, a2b_ref, l1_ref, l2_ref, co_ref,
              al1_ref, al2_ref, alc_ref, *, nk, bk, h):
    k = pl.program_id(1)

    @pl.when(k == 0)
    def _():
        for o in (e1s_ref, e1f_ref, e1a_ref, e2s_ref, e2f_ref, e2a_ref):
            o[...] = jnp.zeros_like(o)

    rows = pl.ds(k * bk, bk)
    y1 = y1_ref[rows, :].astype(BF16)
    y2 = y2_ref[rows, :].astype(BF16)
    a1s = a1s_ref[...].astype(BF16)
    a2s = a2s_ref[...].astype(BF16)
    a1b_ref[...] = a1s
    a2b_ref[...] = a2s
    e1s_ref[...] += jnp.dot(a1s, y1[:, 0 * h:1 * h],
                            preferred_element_type=F32)
    e2s_ref[...] += jnp.dot(a2s, y2[:, 0 * h:1 * h],
                            preferred_element_type=F32)
    e1f_ref[...] += jnp.dot(a1f_ref[...].astype(BF16), y1[:, 1 * h:2 * h],
                            preferred_element_type=F32)
    e1a_ref[...] += jnp.dot(a1a_ref[...].astype(BF16), y1[:, 2 * h:3 * h],
                            preferred_element_type=F32)
    e2f_ref[...] += jnp.dot(a2f_ref[...].astype(BF16), y2[:, 1 * h:2 * h],
                            preferred_element_type=F32)
    e2a_ref[...] += jnp.dot(a2a_ref[...].astype(BF16), y2[:, 2 * h:3 * h],
                            preferred_element_type=F32)

    @pl.when(k == nk - 1)
    def _():
        l1, a1 = _att3(e1s_ref[...], e1f_ref[...], e1a_ref[...],
                       w1_ref[...], u1_ref[...])
        l2, a2 = _att3(e2s_ref[...], e2f_ref[...], e2a_ref[...],
                       w2_ref[...], u2_ref[...])
        s1 = _score(l1, wc_ref[...], uc_ref[...])
        s2 = _score(l2, wc_ref[...], uc_ref[...])
        mx = jnp.maximum(s1, s2)
        x1, x2 = jnp.exp(s1 - mx), jnp.exp(s2 - mx)
        den = x1 + x2
        b1, b2 = x1 / den, x2 / den
        l1_ref[...] = l1
        l2_ref[...] = l2
        co_ref[...] = b1 * l1 + b2 * l2
        al1_ref[...] = a1
        al2_ref[...] = a2
        alc_ref[...] = jnp.concatenate([b1, b2], axis=1)


def _agg(adjs, y1, y2, att_ws, att_us, h, bm, bk):
    n = adjs[0].shape[0]
    nk = n // bk
    kn = y1.shape[1]
    adj_spec = pl.BlockSpec((bm, bk), lambda i, k: (i, k))
    fixed = lambda i, k: (0, 0)
    y_spec = pl.BlockSpec((n, kn), fixed)
    wb = pl.BlockSpec((h, h), fixed)
    ub = pl.BlockSpec((1, h), fixed)
    row_h = pl.BlockSpec((bm, h), lambda i, k: (i, 0))
    return pl.pallas_call(
        functools.partial(_agg_body, nk=nk, bk=bk, h=h),
        grid=(n // bm, nk),
        in_specs=([adj_spec] * 6 + [y_spec, y_spec]
                  + [wb, ub, wb, ub, wb, ub]),
        out_specs=[row_h] * 6 + [adj_spec, adj_spec]
                  + [row_h] * 3
                  + [pl.BlockSpec((bm, 3), lambda i, k: (i, 0)),
                     pl.BlockSpec((bm, 3), lambda i, k: (i, 0)),
                     pl.BlockSpec((bm, 2), lambda i, k: (i, 0))],
        out_shape=[jax.ShapeDtypeStruct((n, h), F32)] * 6
                  + [jax.ShapeDtypeStruct((n, n), BF16)] * 2
                  + [jax.ShapeDtypeStruct((n, h), F32)] * 3
                  + [jax.ShapeDtypeStruct((n, 3), F32),
                     jax.ShapeDtypeStruct((n, 3), F32),
                     jax.ShapeDtypeStruct((n, 2), F32)],
        compiler_params=pltpu.CompilerParams(
            dimension_semantics=("parallel", "arbitrary")),
    )(*adjs, y1, y2,
      att_ws[0], att_us[0].T, att_ws[1], att_us[1].T,
      att_ws[2], att_us[2].T)


def _u4_body(a1_ref, a2_ref, c1_ref, c2_ref, wd1_ref, wd2_ref,
             md1_ref, md2_ref, r1_ref, r2_ref, z2_ref, z1_ref, *, nk, bk, h):
    k = pl.program_id(1)

    @pl.when(k == 0)
    def _():
        z2_ref[...] = jnp.zeros_like(z2_ref)
        z1_ref[...] = jnp.zeros_like(z1_ref)
        r1_ref[...] = jnp.zeros_like(r1_ref)
        r2_ref[...] = jnp.zeros_like(r2_ref)

    rows = pl.ds(k * bk, bk)
    u1 = jnp.dot(a1_ref[...], c1_ref[rows, :].astype(BF16),
                 preferred_element_type=F32)
    u2 = jnp.dot(a2_ref[...], c2_ref[rows, :].astype(BF16),
                 preferred_element_type=F32)
    r1_ref[...] += _dot(u1[:, 0:h], wd1_ref[...])
    r2_ref[...] += _dot(u2[:, 0:h], wd2_ref[...])
    z2_ref[...] += _dot(u1[:, h:], md1_ref[...])
    z1_ref[...] += _dot(u2[:, h:], md2_ref[...])


def _u4(a1b, a2b, c1, c2, wd1, wd2, md1, md2, bm, bk):
    n = c1.shape[0]
    h2 = c1.shape[1]
    h = h2 // 2
    d1 = wd1.shape[1]
    d2 = wd2.shape[1]
    nk = n // bk
    adj_spec = pl.BlockSpec((bm, bk), lambda i, k: (i, k))
    fixed = lambda i, k: (0, 0)
    vec_spec = pl.BlockSpec((n, h2), fixed)
    row_h = pl.BlockSpec((bm, h), lambda i, k: (i, 0))
    return pl.pallas_call(
        functools.partial(_u4_body, nk=nk, bk=bk, h=h),
        grid=(n // bm, nk),
        in_specs=[adj_spec, adj_spec, vec_spec, vec_spec,
                  pl.BlockSpec((h, d1), fixed), pl.BlockSpec((h, d2), fixed),
                  pl.BlockSpec((h, h), fixed), pl.BlockSpec((h, h), fixed)],
        out_specs=[pl.BlockSpec((bm, d1), lambda i, k: (i, 0)),
                   pl.BlockSpec((bm, d2), lambda i, k: (i, 0)),
                   row_h, row_h],
        out_shape=[jax.ShapeDtypeStruct((n, d1), F32),
                   jax.ShapeDtypeStruct((n, d2), F32),
                   jax.ShapeDtypeStruct((n, h), F32),
                   jax.ShapeDtypeStruct((n, h), F32)],
        compiler_params=pltpu.CompilerParams(
            dimension_semantics=("parallel", "arbitrary")),
    )(a1b, a2b, c1, c2, wd1, wd2, md1, md2)


def _xr_body(a1_ref, a2_ref, z2_ref, z1_ref, x2_ref, x1_ref, *, bk):
    k = pl.program_id(1)

    @pl.when(k == 0)
    def _():
        x2_ref[...] = jnp.zeros_like(x2_ref)
        x1_ref[...] = jnp.zeros_like(x1_ref)

    rows = pl.ds(k * bk, bk)
    x2_ref[...] += jnp.dot(a1_ref[...], z2_ref[rows, :].astype(BF16),
                           preferred_element_type=F32)
    x1_ref[...] += jnp.dot(a2_ref[...], z1_ref[rows, :].astype(BF16),
                           preferred_element_type=F32)


def _xr(a1b, a2b, z2, z1, bm, bk):
    n, h = z2.shape
    adj_spec = pl.BlockSpec((bm, bk), lambda i, k: (i, k))
    fixed = lambda i, k: (0, 0)
    vec_spec = pl.BlockSpec((n, h), fixed)
    out_spec = pl.BlockSpec((bm, h), lambda i, k: (i, 0))
    return pl.pallas_call(
        functools.partial(_xr_body, bk=bk),
        grid=(n // bm, n // bk),
        in_specs=[adj_spec, adj_spec, vec_spec, vec_spec],
        out_specs=[out_spec] * 2,
        out_shape=[jax.ShapeDtypeStruct((n, h), F32)] * 2,
        compiler_params=pltpu.CompilerParams(
            dimension_semantics=("parallel", "arbitrary")),
    )(a1b, a2b, z2, z1)


def kernel(features_omics1, features_omics2, adj_spatial_omics1,
           adj_feature_omics1, adj_augmented_omics1, adj_spatial_omics2,
           adj_feature_omics2, adj_augmented_omics2, W_enc1_sp, W_enc1_ft,
           W_enc1_aug, W_enc2_sp, W_enc2_ft, W_enc2_aug, W_dec1, W_dec2,
           att1_w, att1_u, att2_w, att2_u, attc_w, attc_u):
    n = features_omics1.shape[0]
    h = W_enc1_sp.shape[1]
    bm = min(512, n)
    bma = min(256, n)
    bk = min(2048, n)
    bk2 = min(4096, n)

    w1c = jnp.concatenate([W_enc1_sp, W_enc1_ft, W_enc1_aug], axis=1)
    w2c = jnp.concatenate([W_enc2_sp, W_enc2_ft, W_enc2_aug], axis=1)
    y1, y2, md1, md2 = _proj(features_omics1, features_omics2, w1c, w2c,
                             W_dec1, W_enc1_sp, W_dec2, W_enc2_sp, bm=bm)

    (e1s, e1f, e1a, e2s, e2f, e2a, a1b, a2b,
     l1, l2, comb, al1, al2, alc) = _agg(
        (adj_spatial_omics1, adj_feature_omics1, adj_augmented_omics1,
         adj_spatial_omics2, adj_feature_omics2, adj_augmented_omics2),
        y1, y2, (att1_w, att2_w, attc_w), (att1_u, att2_u, attc_u),
        h, bm=bma, bk=bk)

    c1 = jnp.concatenate([comb, l2], axis=1)
    c2 = jnp.concatenate([comb, l1], axis=1)
    rec1, rec2, z2, z1 = _u4(a1b, a2b, c1, c2, W_dec1, W_dec2, md1, md2,
                             bm=bm, bk=bk2)

    x2r, x1r = _xr(a1b, a2b, z2, z1, bm=bm, bk=bk2)

    return (l1, l2, comb, rec1, rec2, x1r, x2r, al1, al2, alc,
            e1s, e1f, e1a, e2s, e2f, e2a)


# speedup vs baseline: 1.8363x; 1.0026x over previous
import functools

import jax
import jax.numpy as jnp
from jax.experimental import pallas as pl
from jax.experimental.pallas import tpu as pltpu

F32 = jnp.float32
BF16 = jnp.bfloat16


def _dot(x, y):
    return jnp.dot(x.astype(BF16), y.astype(BF16), preferred_element_type=F32)


def _proj_body(x1_ref, x2_ref, w1c_ref, w2c_ref, wd1_ref, w1sp_ref,
               wd2_ref, w2sp_ref, y1_ref, y2_ref, md1_ref, md2_ref):
    @pl.when(pl.program_id(0) == 0)
    def _():
        md1_ref[...] = jnp.dot(wd1_ref[...], w1sp_ref[...],
                               preferred_element_type=F32)
        md2_ref[...] = jnp.dot(wd2_ref[...], w2sp_ref[...],
                               preferred_element_type=F32)

    y1_ref[...] = _dot(x1_ref[...], w1c_ref[...])
    y2_ref[...] = _dot(x2_ref[...], w2c_ref[...])


def _proj(x1, x2, w1c, w2c, wd1, w1sp, wd2, w2sp, bm):
    n, d1 = x1.shape
    d2 = x2.shape[1]
    kn = w1c.shape[1]
    h = wd1.shape[0]
    fixed = lambda i: (0, 0)
    return pl.pallas_call(
        _proj_body,
        grid=(n // bm,),
        in_specs=[
            pl.BlockSpec((bm, d1), lambda i: (i, 0)),
            pl.BlockSpec((bm, d2), lambda i: (i, 0)),
            pl.BlockSpec((d1, kn), fixed),
            pl.BlockSpec((d2, kn), fixed),
            pl.BlockSpec((h, d1), fixed),
            pl.BlockSpec((d1, h), fixed),
            pl.BlockSpec((h, d2), fixed),
            pl.BlockSpec((d2, h), fixed),
        ],
        out_specs=[
            pl.BlockSpec((bm, kn), lambda i: (i, 0)),
            pl.BlockSpec((bm, kn), lambda i: (i, 0)),
            pl.BlockSpec((h, h), fixed),
            pl.BlockSpec((h, h), fixed),
        ],
        out_shape=[
            jax.ShapeDtypeStruct((n, kn), F32),
            jax.ShapeDtypeStruct((n, kn), F32),
            jax.ShapeDtypeStruct((h, h), F32),
            jax.ShapeDtypeStruct((h, h), F32),
        ],
        compiler_params=pltpu.CompilerParams(
            dimension_semantics=("arbitrary",)),
    )(x1, x2, w1c, w2c, wd1, w1sp, wd2, w2sp)


def _score(e, w, u_t):
    v = jnp.tanh(jnp.dot(e, w, preferred_element_type=F32))
    return jnp.sum(v * u_t, axis=1, keepdims=True)


def _att3(es, ef, ea, w, u_t):
    ss, sf, sa = _score(es, w, u_t), _score(ef, w, u_t), _score(ea, w, u_t)
    mx = jnp.maximum(jnp.maximum(ss, sf), sa)
    xs, xf, xa = jnp.exp(ss - mx), jnp.exp(sf - mx), jnp.exp(sa - mx)
    den = xs + xf + xa
    als, alf, ala = xs / den, xf / den, xa / den
    l = als * es + alf * ef + ala * ea
    return l, jnp.concatenate([als, alf, ala], axis=1)


def _agg_body(a1s_ref, a1f_ref, a1a_ref, a2s_ref, a2f_ref, a2a_ref,
              y1_ref, y2_ref,
              w1_ref, u1_ref, w2_ref, u2_ref, wc_ref, uc_ref,
              e1s_ref, e1f_ref, e1a_ref, e2s_ref, e2f_ref, e2a_ref,
              a1b_ref, a2b_ref, l1_ref, l2_ref, co_ref,
              al1_ref, al2_ref, alc_ref, *, nk, bk, h):
    k = pl.program_id(1)

    @pl.when(k == 0)
    def _():
        for o in (e1s_ref, e1f_ref, e1a_ref, e2s_ref, e2f_ref, e2a_ref):
            o[...] = jnp.zeros_like(o)

    rows = pl.ds(k * bk, bk)
    y1 = y1_ref[rows, :].astype(BF16)
    y2 = y2_ref[rows, :].astype(BF16)
    a1s = a1s_ref[...].astype(BF16)
    a2s = a2s_ref[...].astype(BF16)
    a1b_ref[...] = a1s
    a2b_ref[...] = a2s
    e1s_ref[...] += jnp.dot(a1s, y1[:, 0 * h:1 * h],
                            preferred_element_type=F32)
    e2s_ref[...] += jnp.dot(a2s, y2[:, 0 * h:1 * h],
                            preferred_element_type=F32)
    e1f_ref[...] += jnp.dot(a1f_ref[...].astype(BF16), y1[:, 1 * h:2 * h],
                            preferred_element_type=F32)
    e1a_ref[...] += jnp.dot(a1a_ref[...].astype(BF16), y1[:, 2 * h:3 * h],
                            preferred_element_type=F32)
    e2f_ref[...] += jnp.dot(a2f_ref[...].astype(BF16), y2[:, 1 * h:2 * h],
                            preferred_element_type=F32)
    e2a_ref[...] += jnp.dot(a2a_ref[...].astype(BF16), y2[:, 2 * h:3 * h],
                            preferred_element_type=F32)

    @pl.when(k == nk - 1)
    def _():
        l1, a1 = _att3(e1s_ref[...], e1f_ref[...], e1a_ref[...],
                       w1_ref[...], u1_ref[...])
        l2, a2 = _att3(e2s_ref[...], e2f_ref[...], e2a_ref[...],
                       w2_ref[...], u2_ref[...])
        s1 = _score(l1, wc_ref[...], uc_ref[...])
        s2 = _score(l2, wc_ref[...], uc_ref[...])
        mx = jnp.maximum(s1, s2)
        x1, x2 = jnp.exp(s1 - mx), jnp.exp(s2 - mx)
        den = x1 + x2
        b1, b2 = x1 / den, x2 / den
        l1_ref[...] = l1
        l2_ref[...] = l2
        co_ref[...] = b1 * l1 + b2 * l2
        al1_ref[...] = a1
        al2_ref[...] = a2
        alc_ref[...] = jnp.concatenate([b1, b2], axis=1)


def _agg(adjs, y1, y2, att_ws, att_us, h, bm, bk):
    n = adjs[0].shape[0]
    nk = n // bk
    kn = y1.shape[1]
    adj_spec = pl.BlockSpec((bm, bk), lambda i, k: (i, k))
    fixed = lambda i, k: (0, 0)
    y_spec = pl.BlockSpec((n, kn), fixed)
    wb = pl.BlockSpec((h, h), fixed)
    ub = pl.BlockSpec((1, h), fixed)
    row_h = pl.BlockSpec((bm, h), lambda i, k: (i, 0))
    return pl.pallas_call(
        functools.partial(_agg_body, nk=nk, bk=bk, h=h),
        grid=(n // bm, nk),
        in_specs=([adj_spec] * 6 + [y_spec, y_spec]
                  + [wb, ub, wb, ub, wb, ub]),
        out_specs=[row_h] * 6 + [adj_spec, adj_spec]
                  + [row_h] * 3
                  + [pl.BlockSpec((bm, 3), lambda i, k: (i, 0)),
                     pl.BlockSpec((bm, 3), lambda i, k: (i, 0)),
                     pl.BlockSpec((bm, 2), lambda i, k: (i, 0))],
        out_shape=[jax.ShapeDtypeStruct((n, h), F32)] * 6
                  + [jax.ShapeDtypeStruct((n, n), BF16)] * 2
                  + [jax.ShapeDtypeStruct((n, h), F32)] * 3
                  + [jax.ShapeDtypeStruct((n, 3), F32),
                     jax.ShapeDtypeStruct((n, 3), F32),
                     jax.ShapeDtypeStruct((n, 2), F32)],
        compiler_params=pltpu.CompilerParams(
            dimension_semantics=("parallel", "arbitrary")),
    )(*adjs, y1, y2,
      att_ws[0], att_us[0].T, att_ws[1], att_us[1].T,
      att_ws[2], att_us[2].T)


def _u4_body(a1_ref, a2_ref, c1_ref, c2_ref, wd1_ref, wd2_ref,
             md1_ref, md2_ref, r1_ref, r2_ref, z2_ref, z1_ref, *, nk, bk, h):
    k = pl.program_id(1)

    @pl.when(k == 0)
    def _():
        z2_ref[...] = jnp.zeros_like(z2_ref)
        z1_ref[...] = jnp.zeros_like(z1_ref)
        r1_ref[...] = jnp.zeros_like(r1_ref)
        r2_ref[...] = jnp.zeros_like(r2_ref)

    rows = pl.ds(k * bk, bk)
    u1 = jnp.dot(a1_ref[...], c1_ref[rows, :].astype(BF16),
                 preferred_element_type=F32)
    u2 = jnp.dot(a2_ref[...], c2_ref[rows, :].astype(BF16),
                 preferred_element_type=F32)
    r1_ref[...] += _dot(u1[:, 0:h], wd1_ref[...])
    r2_ref[...] += _dot(u2[:, 0:h], wd2_ref[...])
    z2_ref[...] += _dot(u1[:, h:], md1_ref[...])
    z1_ref[...] += _dot(u2[:, h:], md2_ref[...])


def _u4(a1b, a2b, c1, c2, wd1, wd2, md1, md2, bm, bk):
    n = c1.shape[0]
    h2 = c1.shape[1]
    h = h2 // 2
    d1 = wd1.shape[1]
    d2 = wd2.shape[1]
    nk = n // bk
    adj_spec = pl.BlockSpec((bm, bk), lambda i, k: (i, k))
    fixed = lambda i, k: (0, 0)
    vec_spec = pl.BlockSpec((n, h2), fixed)
    row_h = pl.BlockSpec((bm, h), lambda i, k: (i, 0))
    return pl.pallas_call(
        functools.partial(_u4_body, nk=nk, bk=bk, h=h),
        grid=(n // bm, nk),
        in_specs=[adj_spec, adj_spec, vec_spec, vec_spec,
                  pl.BlockSpec((h, d1), fixed), pl.BlockSpec((h, d2), fixed),
                  pl.BlockSpec((h, h), fixed), pl.BlockSpec((h, h), fixed)],
        out_specs=[pl.BlockSpec((bm, d1), lambda i, k: (i, 0)),
                   pl.BlockSpec((bm, d2), lambda i, k: (i, 0)),
                   row_h, row_h],
        out_shape=[jax.ShapeDtypeStruct((n, d1), F32),
                   jax.ShapeDtypeStruct((n, d2), F32),
                   jax.ShapeDtypeStruct((n, h), F32),
                   jax.ShapeDtypeStruct((n, h), F32)],
        compiler_params=pltpu.CompilerParams(
            dimension_semantics=("parallel", "arbitrary")),
    )(a1b, a2b, c1, c2, wd1, wd2, md1, md2)


def _xr_body(a1_ref, a2_ref, z2_ref, z1_ref, x2_ref, x1_ref, *, bk):
    k = pl.program_id(1)

    @pl.when(k == 0)
    def _():
        x2_ref[...] = jnp.zeros_like(x2_ref)
        x1_ref[...] = jnp.zeros_like(x1_ref)

    rows = pl.ds(k * bk, bk)
    x2_ref[...] += jnp.dot(a1_ref[...], z2_ref[rows, :].astype(BF16),
                           preferred_element_type=F32)
    x1_ref[...] += jnp.dot(a2_ref[...], z1_ref[rows, :].astype(BF16),
                           preferred_element_type=F32)


def _xr(a1b, a2b, z2, z1, bm, bk):
    n, h = z2.shape
    adj_spec = pl.BlockSpec((bm, bk), lambda i, k: (i, k))
    fixed = lambda i, k: (0, 0)
    vec_spec = pl.BlockSpec((n, h), fixed)
    out_spec = pl.BlockSpec((bm, h), lambda i, k: (i, 0))
    return pl.pallas_call(
        functools.partial(_xr_body, bk=bk),
        grid=(n // bm, n // bk),
        in_specs=[adj_spec, adj_spec, vec_spec, vec_spec],
        out_specs=[out_spec] * 2,
        out_shape=[jax.ShapeDtypeStruct((n, h), F32)] * 2,
        compiler_params=pltpu.CompilerParams(
            dimension_semantics=("parallel", "arbitrary")),
    )(a1b, a2b, z2, z1)


def kernel(features_omics1, features_omics2, adj_spatial_omics1,
           adj_feature_omics1, adj_augmented_omics1, adj_spatial_omics2,
           adj_feature_omics2, adj_augmented_omics2, W_enc1_sp, W_enc1_ft,
           W_enc1_aug, W_enc2_sp, W_enc2_ft, W_enc2_aug, W_dec1, W_dec2,
           att1_w, att1_u, att2_w, att2_u, attc_w, attc_u):
    n = features_omics1.shape[0]
    h = W_enc1_sp.shape[1]
    bm = min(512, n)
    bma = min(256, n)
    bk = min(2048, n)
    bk2 = min(4096, n)

    w1c = jnp.concatenate([W_enc1_sp, W_enc1_ft, W_enc1_aug], axis=1)
    w2c = jnp.concatenate([W_enc2_sp, W_enc2_ft, W_enc2_aug], axis=1)
    y1, y2, md1, md2 = _proj(features_omics1, features_omics2, w1c, w2c,
                             W_dec1, W_enc1_sp, W_dec2, W_enc2_sp,
                             bm=min(1024, n))

    (e1s, e1f, e1a, e2s, e2f, e2a, a1b, a2b,
     l1, l2, comb, al1, al2, alc) = _agg(
        (adj_spatial_omics1, adj_feature_omics1, adj_augmented_omics1,
         adj_spatial_omics2, adj_feature_omics2, adj_augmented_omics2),
        y1, y2, (att1_w, att2_w, attc_w), (att1_u, att2_u, attc_u),
        h, bm=bma, bk=bk)

    c1 = jnp.concatenate([comb, l2], axis=1)
    c2 = jnp.concatenate([comb, l1], axis=1)
    rec1, rec2, z2, z1 = _u4(a1b, a2b, c1, c2, W_dec1, W_dec2, md1, md2,
                             bm=bm, bk=bk2)

    x2r, x1r = _xr(a1b, a2b, z2, z1, bm=min(1024, n), bk=bk2)

    return (l1, l2, comb, rec1, rec2, x1r, x2r, al1, al2, alc,
            e1s, e1f, e1a, e2s, e2f, e2a)
